# baseline (speedup 1.0000x reference)
import functools

import jax
import jax.numpy as jnp
from jax import lax
from jax.experimental import pallas as pl
from jax.experimental.pallas import tpu as pltpu
from jax.experimental.pallas import tpu_sc as plsc

N = 100000
DEG = 16
G = 3
L = 2
B = 4096
ED = 100
ID = 200
OD = 200
S0 = 3
S1 = 5

NC = 2
NS = 16
NW = NC * NS
SEEDS = B // NW

HD = ED // 2
EDP = 104
TS = 512
TGRID = B // TS

_MESH = dict(core_axis_name="c", subcore_axis_name="s")
_SC_PARAMS = pltpu.CompilerParams(use_tc_tiling_on_sc=False)


def _wid():
    return lax.axis_index("s") * NC + lax.axis_index("c")


def _sc_stage_a(nodeids, nbrs_flat, user_flat, base_embed):
    out_type = (
        jax.ShapeDtypeStruct((G, B, EDP), jnp.float32),
        jax.ShapeDtypeStruct((G, B, DEG), jnp.int32),
        jax.ShapeDtypeStruct((B, ID), jnp.float32),
    )
    scratch = [
        pltpu.VMEM((SEEDS,), jnp.int32),
        pltpu.VMEM((SEEDS,), jnp.int32),
        pltpu.VMEM((SEEDS, DEG), jnp.int32),
        pltpu.VMEM((SEEDS, EDP), jnp.float32),
        pltpu.VMEM((SEEDS, ID), jnp.float32),
        pltpu.SemaphoreType.DMA,
    ]

    @functools.partial(pl.kernel, out_type=out_type,
                       mesh=plsc.VectorSubcoreMesh(**_MESH),
                       compiler_params=_SC_PARAMS,
                       scratch_types=scratch)
    def body(nodeids_h, nbrs_h, user_h, base_h, f0_o, rows0_o, base_o,
             seeds_v, idx_v, rows_v, emb_v, base_v, sem):
        wb = _wid() * SEEDS
        pltpu.sync_copy(nodeids_h.at[pl.ds(wb, SEEDS)], seeds_v)
        pltpu.async_copy(base_h.at[seeds_v], base_v, sem).wait()
        pltpu.sync_copy(base_v, base_o.at[pl.ds(wb, SEEDS)])
        for g in range(G):
            for c in range(SEEDS // 16):
                idx_v[pl.ds(c * 16, 16)] = seeds_v[pl.ds(c * 16, 16)] + g * N
            pltpu.async_copy(nbrs_h.at[idx_v], rows_v, sem).wait()
            pltpu.sync_copy(rows_v, rows0_o.at[g, pl.ds(wb, SEEDS)])
            pltpu.async_copy(user_h.at[idx_v], emb_v, sem).wait()
            pltpu.sync_copy(emb_v, f0_o.at[g, pl.ds(wb, SEEDS)])

    return body(nodeids, nbrs_flat, user_flat, base_embed)


def _sc_stage_b(nbrs_flat, user_flat, cur1):
    T = G * S0
    out_type = (
        jax.ShapeDtypeStruct((T, B, EDP), jnp.float32),
        jax.ShapeDtypeStruct((T, B, DEG), jnp.int32),
    )
    scratch = [
        pltpu.VMEM((SEEDS,), jnp.int32),
        pltpu.VMEM((SEEDS, DEG), jnp.int32),
        pltpu.VMEM((SEEDS, EDP), jnp.float32),
        pltpu.SemaphoreType.DMA,
    ]

    @functools.partial(pl.kernel, out_type=out_type,
                       mesh=plsc.VectorSubcoreMesh(**_MESH),
                       compiler_params=_SC_PARAMS,
                       scratch_types=scratch)
    def body(nbrs_h, user_h, cur1_h, f1_o, rows1_o, idx_v, rows_v, emb_v, sem):
        wb = _wid() * SEEDS
        for t in range(T):
            pltpu.sync_copy(cur1_h.at[t, pl.ds(wb, SEEDS)], idx_v)
            pltpu.async_copy(nbrs_h.at[idx_v], rows_v, sem).wait()
            pltpu.sync_copy(rows_v, rows1_o.at[t, pl.ds(wb, SEEDS)])
            pltpu.async_copy(user_h.at[idx_v], emb_v, sem).wait()
            pltpu.sync_copy(emb_v, f1_o.at[t, pl.ds(wb, SEEDS)])

    return body(nbrs_flat, user_flat, cur1)


def _sc_stage_c(user_flat, cur2):
    T = G * S1 * S0
    out_type = jax.ShapeDtypeStruct((T, B, EDP), jnp.float32)
    scratch = [
        pltpu.VMEM((SEEDS,), jnp.int32),
        pltpu.VMEM((SEEDS, EDP), jnp.float32),
        pltpu.SemaphoreType.DMA,
    ]

    @functools.partial(pl.kernel, out_type=out_type,
                       mesh=plsc.VectorSubcoreMesh(**_MESH),
                       compiler_params=_SC_PARAMS,
                       scratch_types=scratch)
    def body(user_h, cur2_h, f2_o, idx_v, emb_v, sem):
        wb = _wid() * SEEDS
        for t in range(T):
            pltpu.sync_copy(cur2_h.at[t, pl.ds(wb, SEEDS)], idx_v)
            pltpu.async_copy(user_h.at[idx_v], emb_v, sem).wait()
            pltpu.sync_copy(emb_v, f2_o.at[t, pl.ds(wb, SEEDS)])

    return body(user_flat, cur2)


def _sel_body(ngroups, rows_per_group):
    del rows_per_group

    def body(rows_r, off_r, out_r):
        for t in range(ngroups):
            if ngroups == G * S0:
                row_idx = t // S0
                bias = (t // S0) * N
            else:
                g = t // (S1 * S0)
                j = t % S0
                row_idx = g * S0 + j
                bias = g * N
            rows = rows_r[row_idx]
            off = off_r[t]
            dcol = lax.broadcasted_iota(jnp.int32, (TS, DEG), 1)
            val = jnp.sum(jnp.where(dcol == off[:, None], rows, 0), axis=1)
            out_r[t] = val + bias
    return body


def _tc_select(rows, off, ngroups, nrows):
    return pl.pallas_call(
        _sel_body(ngroups, nrows),
        grid=(TGRID,),
        in_specs=[
            pl.BlockSpec((nrows, TS, DEG), lambda i: (0, i, 0)),
            pl.BlockSpec((ngroups, TS), lambda i: (0, i)),
        ],
        out_specs=pl.BlockSpec((ngroups, TS), lambda i: (0, i)),
        out_shape=jax.ShapeDtypeStruct((ngroups, B), jnp.int32),
    )(rows, off)


def _dense_body(f0_r, f1_r, f2_r, base_r, et_r,
                wx_r, bx_r, wn_r, bn_r, fcw_r, fcb_r,
                lng_r, lnb_r, wq_r, wk_r, wv_r, wo_r, refl_r, out_r):
    dot = functools.partial(jnp.dot, precision=jax.lax.Precision.HIGHEST,
                            preferred_element_type=jnp.float32)
    spec = []
    for g in range(G):
        wx0a = wx_r[g * L + 0, :HD, :]
        wx0b = wx_r[g * L + 0, HD:, :]
        wn0a = wn_r[g * L + 0, :HD, :]
        wn0b = wn_r[g * L + 0, HD:, :]
        bx0 = bx_r[g * L + 0]
        bn0 = bn_r[g * L + 0]
        wx1a = wx_r[g * L + 1, :HD, :]
        wx1b = wx_r[g * L + 1, HD:, :]
        wn1a = wn_r[g * L + 1, :HD, :]
        wn1b = wn_r[g * L + 1, HD:, :]
        bx1 = bx_r[g * L + 1]
        bn1 = bn_r[g * L + 1]

        f0 = f0_r[g]
        f1 = [f1_r[g * S0 + j] for j in range(S0)]
        h1a, h1b = [], []
        for j in range(S0):
            nb2 = f2_r[(g * S1 + 0) * S0 + j]
            for j2 in range(1, S1):
                nb2 = nb2 + f2_r[(g * S1 + j2) * S0 + j]
            nb2 = nb2 * (1.0 / S1)
            xa = dot(f1[j][:, :HD], wx0a) + dot(f1[j][:, HD:ED], wx0b) + bx0
            xb = dot(nb2[:, :HD], wn0a) + dot(nb2[:, HD:ED], wn0b) + bn0
            h1a.append(jnp.maximum(xa, 0.0))
            h1b.append(jnp.maximum(xb, 0.0))
        nb1 = (f1[0] + f1[1] + f1[2]) * (1.0 / S0)
        xa = dot(f0[:, :HD], wx0a) + dot(f0[:, HD:ED], wx0b) + bx0
        xb = dot(nb1[:, :HD], wn0a) + dot(nb1[:, HD:ED], wn0b) + bn0
        h0a = jnp.maximum(xa, 0.0)
        h0b = jnp.maximum(xb, 0.0)
        nha = (h1a[0] + h1a[1] + h1a[2]) * (1.0 / S0)
        nhb = (h1b[0] + h1b[1] + h1b[2]) * (1.0 / S0)
        ya = dot(h0a, wx1a) + dot(h0b, wx1b) + bx1
        yb = dot(nha, wn1a) + dot(nhb, wn1b) + bn1
        ha = jnp.maximum(ya, 0.0)
        hb = jnp.maximum(yb, 0.0)
        nrm2 = jnp.sum(ha * ha, axis=1, keepdims=True) + \
            jnp.sum(hb * hb, axis=1, keepdims=True)
        inv = 1.0 / jnp.maximum(jnp.sqrt(nrm2), 1e-12)
        fca = fcw_r[g, :HD, :]
        fcb_w = fcw_r[g, HD:, :]
        sg = (dot(ha, fca) + dot(hb, fcb_w)) * inv + fcb_r[g]
        spec.append(sg)

    lng = lng_r[0]
    lnb = lnb_r[0]
    q, k, v = [], [], []
    for g in range(G):
        s = spec[g]
        m = jnp.mean(s, axis=1, keepdims=True)
        var = jnp.mean((s - m) * (s - m), axis=1, keepdims=True)
        qn = (s - m) / jnp.sqrt(var + 1e-6) * lng + lnb
        q.append(dot(qn, wq_r[...]))
        k.append(dot(s, wk_r[...]))
        v.append(dot(s, wv_r[...]))
    scale = 1.0 / jnp.sqrt(jnp.float32(ED))
    o = []
    for qq in range(G):
        lg = [jnp.sum(q[qq] * k[kk], axis=1, keepdims=True) * scale
              for kk in range(G)]
        mx = jnp.maximum(jnp.maximum(lg[0], lg[1]), lg[2])
        e = [jnp.exp(x - mx) for x in lg]
        ssum = e[0] + e[1] + e[2]
        att = [x / ssum for x in e]
        ov = att[0] * v[0] + att[1] * v[1] + att[2] * v[2]
        o.append(dot(ov, wo_r[...]) + spec[qq])

    et = et_r[...]
    acc = jnp.zeros((TS, OD), jnp.float32)
    for g in range(G):
        pr = dot(o[g], refl_r[g])
        acc = acc + jnp.where(et == g, pr, 0.0)
    fin = base_r[...] + acc
    nrm = jnp.sqrt(jnp.sum(fin * fin, axis=1, keepdims=True))
    out_r[...] = fin / jnp.maximum(nrm, 1e-12)


def _dense_call(f0, f1, f2, baser, et2, agg_Wx, agg_bx, agg_Wn, agg_bn,
                fc_W, fc_b, ln_g, ln_b, wq, wk, wv, w_o, reflect):
    def blk(shape, im):
        return pl.BlockSpec(shape, im)

    def full(x):
        nd = x.ndim
        return pl.BlockSpec(x.shape, lambda t, nd=nd: (0,) * nd)

    in_specs = [
        blk((G, TS, EDP), lambda t: (0, t, 0)),
        blk((G * S0, TS, EDP), lambda t: (0, t, 0)),
        blk((G * S1 * S0, TS, EDP), lambda t: (0, t, 0)),
        blk((TS, ID), lambda t: (t, 0)),
        blk((TS, 1), lambda t: (t, 0)),
        full(agg_Wx), full(agg_bx), full(agg_Wn), full(agg_bn),
        full(fc_W), full(fc_b), full(ln_g), full(ln_b),
        full(wq), full(wk), full(wv), full(w_o), full(reflect),
    ]
    return pl.pallas_call(
        _dense_body,
        grid=(TGRID,),
        in_specs=in_specs,
        out_specs=pl.BlockSpec((TS, OD), lambda t: (t, 0)),
        out_shape=jax.ShapeDtypeStruct((B, OD), jnp.float32),
    )(f0, f1, f2, baser, et2, agg_Wx, agg_bx, agg_Wn, agg_bn,
      fc_W, fc_b, ln_g, ln_b, wq, wk, wv, w_o, reflect)


def kernel(nodeids, edgetypes, nbrs, base_embed, user_embed, agg_Wx, agg_bx,
           agg_Wn, agg_bn, fc_W, fc_b, ln_g, ln_b, wq, wk, wv, w_o, reflect):
    skey = jax.random.key(42)
    off0, off1 = [], []
    for g in range(G):
        o0 = jax.random.randint(jax.random.fold_in(skey, g * 100 + 0),
                                (B, S0), 0, DEG, dtype=jnp.int32)
        o1 = jax.random.randint(jax.random.fold_in(skey, g * 100 + 1),
                                (B * S0, S1), 0, DEG, dtype=jnp.int32)
        off0.append(o0.T)
        off1.append(o1.reshape(B, S0, S1).transpose(2, 1, 0))
    off0 = jnp.stack(off0).reshape(G * S0, B)
    off1 = jnp.stack(off1).reshape(G * S1 * S0, B)

    nbrs_flat = nbrs.reshape(G * N, DEG)
    user_flat = jnp.pad(user_embed.reshape(G * N, ED),
                        ((0, 0), (0, EDP - ED)))

    f0, rows0, baser = _sc_stage_a(nodeids, nbrs_flat, user_flat, base_embed)
    cur1 = _tc_select(rows0, off0, G * S0, G)
    f1, rows1 = _sc_stage_b(nbrs_flat, user_flat, cur1)
    cur2 = _tc_select(rows1, off1, G * S1 * S0, G * S0)
    f2 = _sc_stage_c(user_flat, cur2)

    et2 = edgetypes.reshape(B, 1)
    lng2 = ln_g.reshape(1, ED)
    lnb2 = ln_b.reshape(1, ED)
    wx = agg_Wx.reshape(G * L, ED, HD)
    bx = agg_bx.reshape(G * L, HD)
    wn = agg_Wn.reshape(G * L, ED, HD)
    bn = agg_bn.reshape(G * L, HD)

    return _dense_call(f0, f1, f2, baser, et2, wx, bx, wn, bn,
                       fc_W, fc_b, lng2, lnb2, wq, wk, wv, w_o, reflect)


# speedup vs baseline: 1.3020x; 1.3020x over previous
import functools

import jax
import jax.numpy as jnp
from jax import lax
from jax.experimental import pallas as pl
from jax.experimental.pallas import tpu as pltpu
from jax.experimental.pallas import tpu_sc as plsc

N = 100000
DEG = 16
G = 3
L = 2
B = 4096
ED = 100
ID = 200
OD = 200
S0 = 3
S1 = 5

NC = 2
NS = 16
NW = NC * NS
SEEDS = B // NW

HD = ED // 2
EDP = 104
IDP = 200
RCH = 1024
NPAD = 100352
TS = 512
TGRID = B // TS

_MESH = dict(core_axis_name="c", subcore_axis_name="s")
_SC_LIN = pltpu.CompilerParams(use_tc_tiling_on_sc=False)
_SC_TILED = pltpu.CompilerParams(use_tc_tiling_on_sc=True)


def _tc_repack_user(u_t):
    def body(u_r, o_r):
        o_r[:, :ED] = lax.transpose(u_r[0], (1, 0))
        o_r[:, ED:] = jnp.zeros((RCH, EDP - ED), jnp.float32)

    return pl.pallas_call(
        body,
        grid=(G, NPAD // RCH),
        in_specs=[pl.BlockSpec((1, ED, RCH), lambda g, i: (g, 0, i))],
        out_specs=pl.BlockSpec((RCH, EDP),
                               lambda g, i: (g * (NPAD // RCH) + i, 0)),
        out_shape=jax.ShapeDtypeStruct((G * NPAD, EDP), jnp.float32),
    )(u_t)


def _tc_repack_base(b_t):
    def body(b_r, o_r):
        o_r[...] = lax.transpose(b_r[...], (1, 0))

    return pl.pallas_call(
        body,
        grid=(NPAD // RCH,),
        in_specs=[pl.BlockSpec((ID, RCH), lambda i: (0, i))],
        out_specs=pl.BlockSpec((RCH, IDP), lambda i: (i, 0)),
        out_shape=jax.ShapeDtypeStruct((NPAD, IDP), jnp.float32),
    )(b_t)


def _wid():
    return lax.axis_index("s") * NC + lax.axis_index("c")


def _sc_stage_a1(nodeids, nbrs_flat):
    out_type = jax.ShapeDtypeStruct((G, B, DEG), jnp.int32)
    scratch = [
        pltpu.VMEM((SEEDS,), jnp.int32),
        pltpu.VMEM((SEEDS,), jnp.int32),
        pltpu.VMEM((SEEDS, DEG), jnp.int32),
        pltpu.SemaphoreType.DMA,
    ]

    @functools.partial(pl.kernel, out_type=out_type,
                       mesh=plsc.VectorSubcoreMesh(**_MESH),
                       compiler_params=_SC_LIN,
                       scratch_types=scratch)
    def body(nodeids_h, nbrs_h, rows0_o, seeds_v, idx_v, rows_v, sem):
        wb = _wid() * SEEDS
        pltpu.sync_copy(nodeids_h.at[pl.ds(wb, SEEDS)], seeds_v)
        for g in range(G):
            for c in range(SEEDS // 16):
                idx_v[pl.ds(c * 16, 16)] = seeds_v[pl.ds(c * 16, 16)] + g * N
            pltpu.async_copy(nbrs_h.at[idx_v], rows_v, sem).wait()
            pltpu.sync_copy(rows_v, rows0_o.at[g, pl.ds(wb, SEEDS)])

    return body(nodeids, nbrs_flat)


def _sc_stage_a2(nodeids, user128, base256):
    out_type = (
        jax.ShapeDtypeStruct((G, B, EDP), jnp.float32),
        jax.ShapeDtypeStruct((B, IDP), jnp.float32),
    )
    scratch = [
        pltpu.VMEM((SEEDS,), jnp.int32),
        pltpu.VMEM((SEEDS,), jnp.int32),
        pltpu.VMEM((SEEDS, EDP), jnp.float32),
        pltpu.VMEM((SEEDS, IDP), jnp.float32),
        pltpu.SemaphoreType.DMA,
    ]

    @functools.partial(pl.kernel, out_type=out_type,
                       mesh=plsc.VectorSubcoreMesh(**_MESH),
                       compiler_params=_SC_LIN,
                       scratch_types=scratch)
    def body(nodeids_h, user_h, base_h, f0_o, base_o,
             seeds_v, idx_v, emb_v, base_v, sem):
        wb = _wid() * SEEDS
        pltpu.sync_copy(nodeids_h.at[pl.ds(wb, SEEDS)], seeds_v)
        pltpu.async_copy(base_h.at[seeds_v], base_v, sem).wait()
        pltpu.sync_copy(base_v, base_o.at[pl.ds(wb, SEEDS)])
        for g in range(G):
            for c in range(SEEDS // 16):
                idx_v[pl.ds(c * 16, 16)] = seeds_v[pl.ds(c * 16, 16)] + g * NPAD
            pltpu.async_copy(user_h.at[idx_v], emb_v, sem).wait()
            pltpu.sync_copy(emb_v, f0_o.at[g, pl.ds(wb, SEEDS)])

    return body(nodeids, user128, base256)


def _sc_stage_b1(nbrs_flat, cur1):
    T = G * S0
    out_type = jax.ShapeDtypeStruct((T, B, DEG), jnp.int32)
    scratch = [
        pltpu.VMEM((SEEDS,), jnp.int32),
        pltpu.VMEM((SEEDS, DEG), jnp.int32),
        pltpu.SemaphoreType.DMA,
    ]

    @functools.partial(pl.kernel, out_type=out_type,
                       mesh=plsc.VectorSubcoreMesh(**_MESH),
                       compiler_params=_SC_LIN,
                       scratch_types=scratch)
    def body(nbrs_h, cur1_h, rows1_o, idx_v, rows_v, sem):
        wb = _wid() * SEEDS
        for t in range(T):
            pltpu.sync_copy(cur1_h.at[t, pl.ds(wb, SEEDS)], idx_v)
            g = t // S0
            if g:
                for c in range(SEEDS // 16):
                    idx_v[pl.ds(c * 16, 16)] = (idx_v[pl.ds(c * 16, 16)]
                                                - g * (NPAD - N))
            pltpu.async_copy(nbrs_h.at[idx_v], rows_v, sem).wait()
            pltpu.sync_copy(rows_v, rows1_o.at[t, pl.ds(wb, SEEDS)])

    return body(nbrs_flat, cur1)


def _sc_stage_b2(user128, cur1):
    T = G * S0
    out_type = jax.ShapeDtypeStruct((T, B, EDP), jnp.float32)
    scratch = [
        pltpu.VMEM((SEEDS,), jnp.int32),
        pltpu.VMEM((SEEDS, EDP), jnp.float32),
        pltpu.SemaphoreType.DMA,
    ]

    @functools.partial(pl.kernel, out_type=out_type,
                       mesh=plsc.VectorSubcoreMesh(**_MESH),
                       compiler_params=_SC_LIN,
                       scratch_types=scratch)
    def body(user_h, cur1_h, f1_o, idx_v, emb_v, sem):
        wb = _wid() * SEEDS
        for t in range(T):
            pltpu.sync_copy(cur1_h.at[t, pl.ds(wb, SEEDS)], idx_v)
            pltpu.async_copy(user_h.at[idx_v], emb_v, sem).wait()
            pltpu.sync_copy(emb_v, f1_o.at[t, pl.ds(wb, SEEDS)])

    return body(user128, cur1)


def _sc_stage_c(user128, cur2):
    T = G * S1 * S0
    out_type = jax.ShapeDtypeStruct((T, B, EDP), jnp.float32)
    scratch = [
        pltpu.VMEM((SEEDS,), jnp.int32),
        pltpu.VMEM((SEEDS, EDP), jnp.float32),
        pltpu.SemaphoreType.DMA,
    ]

    @functools.partial(pl.kernel, out_type=out_type,
                       mesh=plsc.VectorSubcoreMesh(**_MESH),
                       compiler_params=_SC_LIN,
                       scratch_types=scratch)
    def body(user_h, cur2_h, f2_o, idx_v, emb_v, sem):
        wb = _wid() * SEEDS
        for t in range(T):
            pltpu.sync_copy(cur2_h.at[t, pl.ds(wb, SEEDS)], idx_v)
            pltpu.async_copy(user_h.at[idx_v], emb_v, sem).wait()
            pltpu.sync_copy(emb_v, f2_o.at[t, pl.ds(wb, SEEDS)])

    return body(user128, cur2)


def _sel_body(ngroups, rows_per_group):
    del rows_per_group

    def body(rows_r, off_r, out_r):
        for t in range(ngroups):
            if ngroups == G * S0:
                row_idx = t // S0
                bias = (t // S0) * NPAD
            else:
                g = t // (S1 * S0)
                j = t % S0
                row_idx = g * S0 + j
                bias = g * NPAD
            rows = rows_r[row_idx]
            off = off_r[t]
            dcol = lax.broadcasted_iota(jnp.int32, (TS, DEG), 1)
            val = jnp.sum(jnp.where(dcol == off[:, None], rows, 0), axis=1)
            out_r[t] = val + bias
    return body


def _tc_select(rows, off, ngroups, nrows):
    return pl.pallas_call(
        _sel_body(ngroups, nrows),
        grid=(TGRID,),
        in_specs=[
            pl.BlockSpec((nrows, TS, DEG), lambda i: (0, i, 0)),
            pl.BlockSpec((ngroups, TS), lambda i: (0, i)),
        ],
        out_specs=pl.BlockSpec((ngroups, TS), lambda i: (0, i)),
        out_shape=jax.ShapeDtypeStruct((ngroups, B), jnp.int32),
    )(rows, off)


def _dense_body(f0_r, f1_r, f2_r, base_r, et_r,
                wx_r, bx_r, wn_r, bn_r, fcw_r, fcb_r,
                lng_r, lnb_r, wq_r, wk_r, wv_r, wo_r, refl_r, out_r):
    dot = functools.partial(jnp.dot, precision=jax.lax.Precision.HIGHEST,
                            preferred_element_type=jnp.float32)
    spec = []
    for g in range(G):
        wx0a = wx_r[g * L + 0, :HD, :]
        wx0b = wx_r[g * L + 0, HD:, :]
        wn0a = wn_r[g * L + 0, :HD, :]
        wn0b = wn_r[g * L + 0, HD:, :]
        bx0 = bx_r[g * L + 0]
        bn0 = bn_r[g * L + 0]
        wx1a = wx_r[g * L + 1, :HD, :]
        wx1b = wx_r[g * L + 1, HD:, :]
        wn1a = wn_r[g * L + 1, :HD, :]
        wn1b = wn_r[g * L + 1, HD:, :]
        bx1 = bx_r[g * L + 1]
        bn1 = bn_r[g * L + 1]

        f0 = f0_r[g]
        f1 = [f1_r[g * S0 + j] for j in range(S0)]
        h1a, h1b = [], []
        for j in range(S0):
            nb2 = f2_r[(g * S1 + 0) * S0 + j]
            for j2 in range(1, S1):
                nb2 = nb2 + f2_r[(g * S1 + j2) * S0 + j]
            nb2 = nb2 * (1.0 / S1)
            xa = dot(f1[j][:, :HD], wx0a) + dot(f1[j][:, HD:ED], wx0b) + bx0
            xb = dot(nb2[:, :HD], wn0a) + dot(nb2[:, HD:ED], wn0b) + bn0
            h1a.append(jnp.maximum(xa, 0.0))
            h1b.append(jnp.maximum(xb, 0.0))
        nb1 = (f1[0] + f1[1] + f1[2]) * (1.0 / S0)
        xa = dot(f0[:, :HD], wx0a) + dot(f0[:, HD:ED], wx0b) + bx0
        xb = dot(nb1[:, :HD], wn0a) + dot(nb1[:, HD:ED], wn0b) + bn0
        h0a = jnp.maximum(xa, 0.0)
        h0b = jnp.maximum(xb, 0.0)
        nha = (h1a[0] + h1a[1] + h1a[2]) * (1.0 / S0)
        nhb = (h1b[0] + h1b[1] + h1b[2]) * (1.0 / S0)
        ya = dot(h0a, wx1a) + dot(h0b, wx1b) + bx1
        yb = dot(nha, wn1a) + dot(nhb, wn1b) + bn1
        ha = jnp.maximum(ya, 0.0)
        hb = jnp.maximum(yb, 0.0)
        nrm2 = jnp.sum(ha * ha, axis=1, keepdims=True) + \
            jnp.sum(hb * hb, axis=1, keepdims=True)
        inv = 1.0 / jnp.maximum(jnp.sqrt(nrm2), 1e-12)
        fca = fcw_r[g, :HD, :]
        fcb_w = fcw_r[g, HD:, :]
        sg = (dot(ha, fca) + dot(hb, fcb_w)) * inv + fcb_r[g]
        spec.append(sg)

    lng = lng_r[0]
    lnb = lnb_r[0]
    q, k, v = [], [], []
    for g in range(G):
        s = spec[g]
        m = jnp.mean(s, axis=1, keepdims=True)
        var = jnp.mean((s - m) * (s - m), axis=1, keepdims=True)
        qn = (s - m) / jnp.sqrt(var + 1e-6) * lng + lnb
        q.append(dot(qn, wq_r[...]))
        k.append(dot(s, wk_r[...]))
        v.append(dot(s, wv_r[...]))
    scale = 1.0 / jnp.sqrt(jnp.float32(ED))
    o = []
    for qq in range(G):
        lg = [jnp.sum(q[qq] * k[kk], axis=1, keepdims=True) * scale
              for kk in range(G)]
        mx = jnp.maximum(jnp.maximum(lg[0], lg[1]), lg[2])
        e = [jnp.exp(x - mx) for x in lg]
        ssum = e[0] + e[1] + e[2]
        att = [x / ssum for x in e]
        ov = att[0] * v[0] + att[1] * v[1] + att[2] * v[2]
        o.append(dot(ov, wo_r[...]) + spec[qq])

    et = et_r[...]
    acc = jnp.zeros((TS, OD), jnp.float32)
    for g in range(G):
        pr = dot(o[g], refl_r[g])
        acc = acc + jnp.where(et == g, pr, 0.0)
    fin = base_r[:, :ID] + acc
    nrm = jnp.sqrt(jnp.sum(fin * fin, axis=1, keepdims=True))
    out_r[...] = fin / jnp.maximum(nrm, 1e-12)


def _dense_call(f0, f1, f2, baser, et2, agg_Wx, agg_bx, agg_Wn, agg_bn,
                fc_W, fc_b, ln_g, ln_b, wq, wk, wv, w_o, reflect):
    def blk(shape, im):
        return pl.BlockSpec(shape, im)

    def full(x):
        nd = x.ndim
        return pl.BlockSpec(x.shape, lambda t, nd=nd: (0,) * nd)

    in_specs = [
        blk((G, TS, EDP), lambda t: (0, t, 0)),
        blk((G * S0, TS, EDP), lambda t: (0, t, 0)),
        blk((G * S1 * S0, TS, EDP), lambda t: (0, t, 0)),
        blk((TS, IDP), lambda t: (t, 0)),
        blk((TS, 1), lambda t: (t, 0)),
        full(agg_Wx), full(agg_bx), full(agg_Wn), full(agg_bn),
        full(fc_W), full(fc_b), full(ln_g), full(ln_b),
        full(wq), full(wk), full(wv), full(w_o), full(reflect),
    ]
    return pl.pallas_call(
        _dense_body,
        grid=(TGRID,),
        in_specs=in_specs,
        out_specs=pl.BlockSpec((TS, OD), lambda t: (t, 0)),
        out_shape=jax.ShapeDtypeStruct((B, OD), jnp.float32),
    )(f0, f1, f2, baser, et2, agg_Wx, agg_bx, agg_Wn, agg_bn,
      fc_W, fc_b, ln_g, ln_b, wq, wk, wv, w_o, reflect)


def kernel(nodeids, edgetypes, nbrs, base_embed, user_embed, agg_Wx, agg_bx,
           agg_Wn, agg_bn, fc_W, fc_b, ln_g, ln_b, wq, wk, wv, w_o, reflect):
    skey = jax.random.key(42)
    off0, off1 = [], []
    for g in range(G):
        o0 = jax.random.randint(jax.random.fold_in(skey, g * 100 + 0),
                                (B, S0), 0, DEG, dtype=jnp.int32)
        o1 = jax.random.randint(jax.random.fold_in(skey, g * 100 + 1),
                                (B * S0, S1), 0, DEG, dtype=jnp.int32)
        off0.append(o0.T)
        off1.append(o1.reshape(B, S0, S1).transpose(2, 1, 0))
    off0 = jnp.stack(off0).reshape(G * S0, B)
    off1 = jnp.stack(off1).reshape(G * S1 * S0, B)

    nbrs_flat = nbrs.reshape(G * N, DEG)
    user128 = _tc_repack_user(jnp.transpose(user_embed, (0, 2, 1)))
    base256 = _tc_repack_base(jnp.transpose(base_embed, (1, 0)))

    rows0 = _sc_stage_a1(nodeids, nbrs_flat)
    f0, baser = _sc_stage_a2(nodeids, user128, base256)
    cur1 = _tc_select(rows0, off0, G * S0, G)
    rows1 = _sc_stage_b1(nbrs_flat, cur1)
    f1 = _sc_stage_b2(user128, cur1)
    cur2 = _tc_select(rows1, off1, G * S1 * S0, G * S0)
    f2 = _sc_stage_c(user128, cur2)

    et2 = edgetypes.reshape(B, 1)
    lng2 = ln_g.reshape(1, ED)
    lnb2 = ln_b.reshape(1, ED)
    wx = agg_Wx.reshape(G * L, ED, HD)
    bx = agg_bx.reshape(G * L, HD)
    wn = agg_Wn.reshape(G * L, ED, HD)
    bn = agg_bn.reshape(G * L, HD)

    return _dense_call(f0, f1, f2, baser, et2, wx, bx, wn, bn,
                       fc_W, fc_b, lng2, lnb2, wq, wk, wv, w_o, reflect)


# speedup vs baseline: 1.3887x; 1.0666x over previous
import functools

import jax
import jax.numpy as jnp
from jax import lax
from jax.experimental import pallas as pl
from jax.experimental.pallas import tpu as pltpu
from jax.experimental.pallas import tpu_sc as plsc

N = 100000
DEG = 16
G = 3
L = 2
B = 4096
ED = 100
ID = 200
OD = 200
S0 = 3
S1 = 5

NC = 2
NS = 16
NW = NC * NS
SEEDS = B // NW

HD = ED // 2
EDP = 128
IDP = 200
RCH = 1024
NPAD = 100352
TS = 512
TGRID = B // TS

_MESH = dict(core_axis_name="c", subcore_axis_name="s")
_SC_LIN = pltpu.CompilerParams(use_tc_tiling_on_sc=False)
_SC_TILED = pltpu.CompilerParams(use_tc_tiling_on_sc=True)


def _tc_repack_user(u_t):
    def body(u_r, o_r):
        o_r[:, :ED] = lax.transpose(u_r[0], (1, 0))
        o_r[:, ED:] = jnp.zeros((RCH, EDP - ED), jnp.float32)

    return pl.pallas_call(
        body,
        grid=(G, NPAD // RCH),
        in_specs=[pl.BlockSpec((1, ED, RCH), lambda g, i: (g, 0, i))],
        out_specs=pl.BlockSpec((RCH, EDP),
                               lambda g, i: (g * (NPAD // RCH) + i, 0)),
        out_shape=jax.ShapeDtypeStruct((G * NPAD, EDP), jnp.float32),
    )(u_t)


def _tc_repack_base(b_t):
    def body(b_r, o_r):
        o_r[...] = lax.transpose(b_r[...], (1, 0))

    return pl.pallas_call(
        body,
        grid=(NPAD // RCH,),
        in_specs=[pl.BlockSpec((ID, RCH), lambda i: (0, i))],
        out_specs=pl.BlockSpec((RCH, IDP), lambda i: (i, 0)),
        out_shape=jax.ShapeDtypeStruct((NPAD, IDP), jnp.float32),
    )(b_t)


def _wid():
    return lax.axis_index("s") * NC + lax.axis_index("c")


def _sc_stage_a1(nodeids, nbrs_flat):
    out_type = jax.ShapeDtypeStruct((G, B, DEG), jnp.int32)
    scratch = [
        pltpu.VMEM((SEEDS,), jnp.int32),
        pltpu.VMEM((SEEDS,), jnp.int32),
        pltpu.VMEM((SEEDS, DEG), jnp.int32),
        pltpu.SemaphoreType.DMA,
    ]

    @functools.partial(pl.kernel, out_type=out_type,
                       mesh=plsc.VectorSubcoreMesh(**_MESH),
                       compiler_params=_SC_LIN,
                       scratch_types=scratch)
    def body(nodeids_h, nbrs_h, rows0_o, seeds_v, idx_v, rows_v, sem):
        wb = _wid() * SEEDS
        pltpu.sync_copy(nodeids_h.at[pl.ds(wb, SEEDS)], seeds_v)
        for g in range(G):
            for c in range(SEEDS // 16):
                idx_v[pl.ds(c * 16, 16)] = seeds_v[pl.ds(c * 16, 16)] + g * N
            pltpu.async_copy(nbrs_h.at[idx_v], rows_v, sem).wait()
            pltpu.sync_copy(rows_v, rows0_o.at[g, pl.ds(wb, SEEDS)])

    return body(nodeids, nbrs_flat)


def _sc_stage_a2(nodeids, user128, base256):
    out_type = (
        jax.ShapeDtypeStruct((G, B, EDP), jnp.float32),
        jax.ShapeDtypeStruct((B, IDP), jnp.float32),
    )
    scratch = [
        pltpu.VMEM((SEEDS,), jnp.int32),
        pltpu.VMEM((SEEDS,), jnp.int32),
        pltpu.VMEM((SEEDS, EDP), jnp.float32),
        pltpu.VMEM((SEEDS, IDP), jnp.float32),
        pltpu.SemaphoreType.DMA,
    ]

    @functools.partial(pl.kernel, out_type=out_type,
                       mesh=plsc.VectorSubcoreMesh(**_MESH),
                       compiler_params=_SC_LIN,
                       scratch_types=scratch)
    def body(nodeids_h, user_h, base_h, f0_o, base_o,
             seeds_v, idx_v, emb_v, base_v, sem):
        wb = _wid() * SEEDS
        pltpu.sync_copy(nodeids_h.at[pl.ds(wb, SEEDS)], seeds_v)
        pltpu.async_copy(base_h.at[seeds_v], base_v, sem).wait()
        pltpu.sync_copy(base_v, base_o.at[pl.ds(wb, SEEDS)])
        for g in range(G):
            for c in range(SEEDS // 16):
                idx_v[pl.ds(c * 16, 16)] = seeds_v[pl.ds(c * 16, 16)] + g * NPAD
            pltpu.async_copy(user_h.at[idx_v], emb_v, sem).wait()
            pltpu.sync_copy(emb_v, f0_o.at[g, pl.ds(wb, SEEDS)])

    return body(nodeids, user128, base256)


def _sc_stage_b1(nbrs_flat, cur1):
    T = G * S0
    out_type = jax.ShapeDtypeStruct((T, B, DEG), jnp.int32)
    scratch = [
        pltpu.VMEM((SEEDS,), jnp.int32),
        pltpu.VMEM((SEEDS, DEG), jnp.int32),
        pltpu.SemaphoreType.DMA,
    ]

    @functools.partial(pl.kernel, out_type=out_type,
                       mesh=plsc.VectorSubcoreMesh(**_MESH),
                       compiler_params=_SC_LIN,
                       scratch_types=scratch)
    def body(nbrs_h, cur1_h, rows1_o, idx_v, rows_v, sem):
        wb = _wid() * SEEDS
        for t in range(T):
            pltpu.sync_copy(cur1_h.at[t, pl.ds(wb, SEEDS)], idx_v)
            g = t // S0
            if g:
                for c in range(SEEDS // 16):
                    idx_v[pl.ds(c * 16, 16)] = (idx_v[pl.ds(c * 16, 16)]
                                                - g * (NPAD - N))
            pltpu.async_copy(nbrs_h.at[idx_v], rows_v, sem).wait()
            pltpu.sync_copy(rows_v, rows1_o.at[t, pl.ds(wb, SEEDS)])

    return body(nbrs_flat, cur1)


def _sc_stage_b2(user128, cur1):
    T = G * S0
    out_type = jax.ShapeDtypeStruct((T, B, EDP), jnp.float32)
    scratch = [
        pltpu.VMEM((SEEDS,), jnp.int32),
        pltpu.VMEM((SEEDS, EDP), jnp.float32),
        pltpu.SemaphoreType.DMA,
    ]

    @functools.partial(pl.kernel, out_type=out_type,
                       mesh=plsc.VectorSubcoreMesh(**_MESH),
                       compiler_params=_SC_LIN,
                       scratch_types=scratch)
    def body(user_h, cur1_h, f1_o, idx_v, emb_v, sem):
        wb = _wid() * SEEDS
        for t in range(T):
            pltpu.sync_copy(cur1_h.at[t, pl.ds(wb, SEEDS)], idx_v)
            pltpu.async_copy(user_h.at[idx_v], emb_v, sem).wait()
            pltpu.sync_copy(emb_v, f1_o.at[t, pl.ds(wb, SEEDS)])

    return body(user128, cur1)


def _sc_stage_c(user128, cur2):
    T = G * S1 * S0
    out_type = jax.ShapeDtypeStruct((T, B, EDP), jnp.float32)
    scratch = [
        pltpu.VMEM((SEEDS,), jnp.int32),
        pltpu.VMEM((SEEDS, EDP), jnp.float32),
        pltpu.SemaphoreType.DMA,
    ]

    @functools.partial(pl.kernel, out_type=out_type,
                       mesh=plsc.VectorSubcoreMesh(**_MESH),
                       compiler_params=_SC_LIN,
                       scratch_types=scratch)
    def body(user_h, cur2_h, f2_o, idx_v, emb_v, sem):
        wb = _wid() * SEEDS
        for t in range(T):
            pltpu.sync_copy(cur2_h.at[t, pl.ds(wb, SEEDS)], idx_v)
            pltpu.async_copy(user_h.at[idx_v], emb_v, sem).wait()
            pltpu.sync_copy(emb_v, f2_o.at[t, pl.ds(wb, SEEDS)])

    return body(user128, cur2)


def _sel_body(ngroups):
    TSP = TS // 8

    def body(rows_r, off_r, out_r):
        dcol = lax.broadcasted_iota(jnp.int32, (TSP, DEG * 8), 1)
        for t in range(ngroups):
            if ngroups == G * S0:
                row_idx = t // S0
                bias = (t // S0) * NPAD
            else:
                g = t // (S1 * S0)
                j = t % S0
                row_idx = g * S0 + j
                bias = g * NPAD
            rows = rows_r[row_idx]
            vals = []
            for j8 in range(8):
                col = off_r[t, :, j8:j8 + 1] + (j8 * DEG)
                vals.append(
                    jnp.sum(jnp.where(dcol == col, rows, 0), axis=1))
            out_r[t] = jnp.stack(vals, axis=1) + bias
    return body


def _tc_select(rows_packed, off_packed, ngroups, nrows):
    TSP = TS // 8
    out = pl.pallas_call(
        _sel_body(ngroups),
        grid=(TGRID,),
        in_specs=[
            pl.BlockSpec((nrows, TSP, DEG * 8), lambda i: (0, i, 0)),
            pl.BlockSpec((ngroups, TSP, 8), lambda i: (0, i, 0)),
        ],
        out_specs=pl.BlockSpec((ngroups, TSP, 8), lambda i: (0, i, 0)),
        out_shape=jax.ShapeDtypeStruct((ngroups, B // 8, 8), jnp.int32),
    )(rows_packed, off_packed)
    return out.reshape(ngroups, B)


def _dense_body(f0_r, f1_r, f2_r, base_r, et_r,
                wx_r, bx_r, wn_r, bn_r, fcw_r, fcb_r,
                lng_r, lnb_r, wq_r, wk_r, wv_r, wo_r, refl_r, out_r):
    dot = functools.partial(jnp.dot, precision=jax.lax.Precision.HIGHEST,
                            preferred_element_type=jnp.float32)
    spec = []
    for g in range(G):
        wx0a = wx_r[g * L + 0, :HD, :]
        wx0b = wx_r[g * L + 0, HD:, :]
        wn0a = wn_r[g * L + 0, :HD, :]
        wn0b = wn_r[g * L + 0, HD:, :]
        bx0 = bx_r[g * L + 0]
        bn0 = bn_r[g * L + 0]
        wx1a = wx_r[g * L + 1, :HD, :]
        wx1b = wx_r[g * L + 1, HD:, :]
        wn1a = wn_r[g * L + 1, :HD, :]
        wn1b = wn_r[g * L + 1, HD:, :]
        bx1 = bx_r[g * L + 1]
        bn1 = bn_r[g * L + 1]

        f0 = f0_r[g]
        f1 = [f1_r[g * S0 + j] for j in range(S0)]
        h1a, h1b = [], []
        for j in range(S0):
            nb2 = f2_r[(g * S1 + 0) * S0 + j]
            for j2 in range(1, S1):
                nb2 = nb2 + f2_r[(g * S1 + j2) * S0 + j]
            nb2 = nb2 * (1.0 / S1)
            xa = dot(f1[j][:, :HD], wx0a) + dot(f1[j][:, HD:ED], wx0b) + bx0
            xb = dot(nb2[:, :HD], wn0a) + dot(nb2[:, HD:ED], wn0b) + bn0
            h1a.append(jnp.maximum(xa, 0.0))
            h1b.append(jnp.maximum(xb, 0.0))
        nb1 = (f1[0] + f1[1] + f1[2]) * (1.0 / S0)
        xa = dot(f0[:, :HD], wx0a) + dot(f0[:, HD:ED], wx0b) + bx0
        xb = dot(nb1[:, :HD], wn0a) + dot(nb1[:, HD:ED], wn0b) + bn0
        h0a = jnp.maximum(xa, 0.0)
        h0b = jnp.maximum(xb, 0.0)
        nha = (h1a[0] + h1a[1] + h1a[2]) * (1.0 / S0)
        nhb = (h1b[0] + h1b[1] + h1b[2]) * (1.0 / S0)
        ya = dot(h0a, wx1a) + dot(h0b, wx1b) + bx1
        yb = dot(nha, wn1a) + dot(nhb, wn1b) + bn1
        ha = jnp.maximum(ya, 0.0)
        hb = jnp.maximum(yb, 0.0)
        nrm2 = jnp.sum(ha * ha, axis=1, keepdims=True) + \
            jnp.sum(hb * hb, axis=1, keepdims=True)
        inv = 1.0 / jnp.maximum(jnp.sqrt(nrm2), 1e-12)
        fca = fcw_r[g, :HD, :]
        fcb_w = fcw_r[g, HD:, :]
        sg = (dot(ha, fca) + dot(hb, fcb_w)) * inv + fcb_r[g]
        spec.append(sg)

    lng = lng_r[0]
    lnb = lnb_r[0]
    q, k, v = [], [], []
    for g in range(G):
        s = spec[g]
        m = jnp.mean(s, axis=1, keepdims=True)
        var = jnp.mean((s - m) * (s - m), axis=1, keepdims=True)
        qn = (s - m) / jnp.sqrt(var + 1e-6) * lng + lnb
        q.append(dot(qn, wq_r[...]))
        k.append(dot(s, wk_r[...]))
        v.append(dot(s, wv_r[...]))
    scale = 1.0 / jnp.sqrt(jnp.float32(ED))
    o = []
    for qq in range(G):
        lg = [jnp.sum(q[qq] * k[kk], axis=1, keepdims=True) * scale
              for kk in range(G)]
        mx = jnp.maximum(jnp.maximum(lg[0], lg[1]), lg[2])
        e = [jnp.exp(x - mx) for x in lg]
        ssum = e[0] + e[1] + e[2]
        att = [x / ssum for x in e]
        ov = att[0] * v[0] + att[1] * v[1] + att[2] * v[2]
        o.append(dot(ov, wo_r[...]) + spec[qq])

    et = et_r[...]
    acc = jnp.zeros((TS, OD), jnp.float32)
    for g in range(G):
        pr = dot(o[g], refl_r[g])
        acc = acc + jnp.where(et == g, pr, 0.0)
    fin = base_r[:, :ID] + acc
    nrm = jnp.sqrt(jnp.sum(fin * fin, axis=1, keepdims=True))
    out_r[...] = fin / jnp.maximum(nrm, 1e-12)


def _dense_call(f0, f1, f2, baser, et2, agg_Wx, agg_bx, agg_Wn, agg_bn,
                fc_W, fc_b, ln_g, ln_b, wq, wk, wv, w_o, reflect):
    def blk(shape, im):
        return pl.BlockSpec(shape, im)

    def full(x):
        nd = x.ndim
        return pl.BlockSpec(x.shape, lambda t, nd=nd: (0,) * nd)

    in_specs = [
        blk((G, TS, EDP), lambda t: (0, t, 0)),
        blk((G * S0, TS, EDP), lambda t: (0, t, 0)),
        blk((G * S1 * S0, TS, EDP), lambda t: (0, t, 0)),
        blk((TS, IDP), lambda t: (t, 0)),
        blk((TS, 1), lambda t: (t, 0)),
        full(agg_Wx), full(agg_bx), full(agg_Wn), full(agg_bn),
        full(fc_W), full(fc_b), full(ln_g), full(ln_b),
        full(wq), full(wk), full(wv), full(w_o), full(reflect),
    ]
    return pl.pallas_call(
        _dense_body,
        grid=(TGRID,),
        in_specs=in_specs,
        out_specs=pl.BlockSpec((TS, OD), lambda t: (t, 0)),
        out_shape=jax.ShapeDtypeStruct((B, OD), jnp.float32),
    )(f0, f1, f2, baser, et2, agg_Wx, agg_bx, agg_Wn, agg_bn,
      fc_W, fc_b, ln_g, ln_b, wq, wk, wv, w_o, reflect)


def kernel(nodeids, edgetypes, nbrs, base_embed, user_embed, agg_Wx, agg_bx,
           agg_Wn, agg_bn, fc_W, fc_b, ln_g, ln_b, wq, wk, wv, w_o, reflect):
    skey = jax.random.key(42)
    off0, off1 = [], []
    for g in range(G):
        o0 = jax.random.randint(jax.random.fold_in(skey, g * 100 + 0),
                                (B, S0), 0, DEG, dtype=jnp.int32)
        o1 = jax.random.randint(jax.random.fold_in(skey, g * 100 + 1),
                                (B * S0, S1), 0, DEG, dtype=jnp.int32)
        off0.append(o0.T)
        off1.append(o1.reshape(B, S0, S1).transpose(2, 1, 0))
    off0 = jnp.stack(off0).reshape(G * S0, B // 8, 8)
    off1 = jnp.stack(off1).reshape(G * S1 * S0, B // 8, 8)

    nbrs_flat = nbrs.reshape(G * N, DEG)
    user128 = _tc_repack_user(jnp.transpose(user_embed, (0, 2, 1)))
    base256 = _tc_repack_base(jnp.transpose(base_embed, (1, 0)))

    rows0 = _sc_stage_a1(nodeids, nbrs_flat)
    f0, baser = _sc_stage_a2(nodeids, user128, base256)
    cur1 = _tc_select(rows0.reshape(G, B // 8, DEG * 8),
                      off0, G * S0, G)
    rows1 = _sc_stage_b1(nbrs_flat, cur1)
    f1 = _sc_stage_b2(user128, cur1)
    cur2 = _tc_select(rows1.reshape(G * S0, B // 8, DEG * 8),
                      off1, G * S1 * S0, G * S0)
    f2 = _sc_stage_c(user128, cur2)

    et2 = edgetypes.reshape(B, 1)
    lng2 = ln_g.reshape(1, ED)
    lnb2 = ln_b.reshape(1, ED)
    wx = agg_Wx.reshape(G * L, ED, HD)
    bx = agg_bx.reshape(G * L, HD)
    wn = agg_Wn.reshape(G * L, ED, HD)
    bn = agg_bn.reshape(G * L, HD)

    return _dense_call(f0, f1, f2, baser, et2, wx, bx, wn, bn,
                       fc_W, fc_b, lng2, lnb2, wq, wk, wv, w_o, reflect)


# speedup vs baseline: 1.5436x; 1.1115x over previous
import functools

import jax
import jax.numpy as jnp
from jax import lax
from jax.experimental import pallas as pl
from jax.experimental.pallas import tpu as pltpu
from jax.experimental.pallas import tpu_sc as plsc

N = 100000
DEG = 16
G = 3
L = 2
B = 4096
ED = 100
ID = 200
OD = 200
S0 = 3
S1 = 5

NC = 2
NS = 16
NW = NC * NS
SEEDS = B // NW

HD = ED // 2
EDP = 128
IDP = 256
RCH = 1024
NPAD = 100352
TS = 512
TGRID = B // TS

_MESH = dict(core_axis_name="c", subcore_axis_name="s")
_SC_LIN = pltpu.CompilerParams(use_tc_tiling_on_sc=False)
_SC_TILED = pltpu.CompilerParams(use_tc_tiling_on_sc=True)


def _tc_repack_user(u_t):
    def body(u_r, o_r):
        o_r[:, :ED] = lax.transpose(u_r[0], (1, 0))
        o_r[:, ED:] = jnp.zeros((RCH, EDP - ED), jnp.float32)

    return pl.pallas_call(
        body,
        grid=(G, NPAD // RCH),
        in_specs=[pl.BlockSpec((1, ED, RCH), lambda g, i: (g, 0, i))],
        out_specs=pl.BlockSpec((RCH, EDP),
                               lambda g, i: (g * (NPAD // RCH) + i, 0)),
        out_shape=jax.ShapeDtypeStruct((G * NPAD, EDP), jnp.float32),
    )(u_t)


def _tc_repack_base(b_t):
    def body(b_r, o_r):
        o_r[:, :ID] = lax.transpose(b_r[...], (1, 0))
        o_r[:, ID:] = jnp.zeros((RCH, IDP - ID), jnp.float32)

    return pl.pallas_call(
        body,
        grid=(NPAD // RCH,),
        in_specs=[pl.BlockSpec((ID, RCH), lambda i: (0, i))],
        out_specs=pl.BlockSpec((RCH, IDP), lambda i: (i, 0)),
        out_shape=jax.ShapeDtypeStruct((NPAD, IDP), jnp.float32),
    )(b_t)


def _wid():
    return lax.axis_index("s") * NC + lax.axis_index("c")


def _sc_stage_a1(nodeids, nbrs_flat):
    out_type = jax.ShapeDtypeStruct((G, B, DEG), jnp.int32)
    scratch = [
        pltpu.VMEM((SEEDS,), jnp.int32),
        pltpu.VMEM((SEEDS,), jnp.int32),
        pltpu.VMEM((SEEDS, DEG), jnp.int32),
        pltpu.SemaphoreType.DMA,
    ]

    @functools.partial(pl.kernel, out_type=out_type,
                       mesh=plsc.VectorSubcoreMesh(**_MESH),
                       compiler_params=_SC_LIN,
                       scratch_types=scratch)
    def body(nodeids_h, nbrs_h, rows0_o, seeds_v, idx_v, rows_v, sem):
        wb = _wid() * SEEDS
        pltpu.sync_copy(nodeids_h.at[pl.ds(wb, SEEDS)], seeds_v)
        for g in range(G):
            for c in range(SEEDS // 16):
                idx_v[pl.ds(c * 16, 16)] = seeds_v[pl.ds(c * 16, 16)] + g * N
            pltpu.async_copy(nbrs_h.at[idx_v], rows_v, sem).wait()
            pltpu.sync_copy(rows_v, rows0_o.at[g, pl.ds(wb, SEEDS)])

    return body(nodeids, nbrs_flat)


def _sc_stage_a2(nodeids, user128, base256):
    out_type = (
        jax.ShapeDtypeStruct((G, B, EDP), jnp.float32),
        jax.ShapeDtypeStruct((B, IDP), jnp.float32),
    )
    scratch = [
        pltpu.VMEM((SEEDS,), jnp.int32),
        pltpu.VMEM((SEEDS,), jnp.int32),
        pltpu.VMEM((SEEDS, EDP), jnp.float32),
        pltpu.VMEM((SEEDS, IDP), jnp.float32),
        pltpu.SemaphoreType.DMA,
    ]

    @functools.partial(pl.kernel, out_type=out_type,
                       mesh=plsc.VectorSubcoreMesh(**_MESH),
                       compiler_params=_SC_LIN,
                       scratch_types=scratch)
    def body(nodeids_h, user_h, base_h, f0_o, base_o,
             seeds_v, idx_v, emb_v, base_v, sem):
        wb = _wid() * SEEDS
        pltpu.sync_copy(nodeids_h.at[pl.ds(wb, SEEDS)], seeds_v)
        pltpu.async_copy(base_h.at[seeds_v], base_v, sem).wait()
        pltpu.sync_copy(base_v, base_o.at[pl.ds(wb, SEEDS)])
        for g in range(G):
            for c in range(SEEDS // 16):
                idx_v[pl.ds(c * 16, 16)] = seeds_v[pl.ds(c * 16, 16)] + g * NPAD
            pltpu.async_copy(user_h.at[idx_v], emb_v, sem).wait()
            pltpu.sync_copy(emb_v, f0_o.at[g, pl.ds(wb, SEEDS)])

    return body(nodeids, user128, base256)


def _sc_stage_b1(nbrs_flat, cur1):
    T = G * S0
    out_type = jax.ShapeDtypeStruct((T, B, DEG), jnp.int32)
    scratch = [
        pltpu.VMEM((SEEDS,), jnp.int32),
        pltpu.VMEM((SEEDS, DEG), jnp.int32),
        pltpu.SemaphoreType.DMA,
    ]

    @functools.partial(pl.kernel, out_type=out_type,
                       mesh=plsc.VectorSubcoreMesh(**_MESH),
                       compiler_params=_SC_LIN,
                       scratch_types=scratch)
    def body(nbrs_h, cur1_h, rows1_o, idx_v, rows_v, sem):
        wb = _wid() * SEEDS
        for t in range(T):
            pltpu.sync_copy(cur1_h.at[t, pl.ds(wb, SEEDS)], idx_v)
            g = t // S0
            if g:
                for c in range(SEEDS // 16):
                    idx_v[pl.ds(c * 16, 16)] = (idx_v[pl.ds(c * 16, 16)]
                                                - g * (NPAD - N))
            pltpu.async_copy(nbrs_h.at[idx_v], rows_v, sem).wait()
            pltpu.sync_copy(rows_v, rows1_o.at[t, pl.ds(wb, SEEDS)])

    return body(nbrs_flat, cur1)


def _sc_stage_b2(user128, cur1):
    T = G * S0
    out_type = jax.ShapeDtypeStruct((T, B, EDP), jnp.float32)
    scratch = [
        pltpu.VMEM((T, SEEDS), jnp.int32),
        pltpu.VMEM((SEEDS, EDP), jnp.float32),
        pltpu.VMEM((SEEDS, EDP), jnp.float32),
        pltpu.SemaphoreType.DMA,
        pltpu.SemaphoreType.DMA,
        pltpu.SemaphoreType.DMA,
    ]

    @functools.partial(pl.kernel, out_type=out_type,
                       mesh=plsc.VectorSubcoreMesh(**_MESH),
                       compiler_params=_SC_LIN,
                       scratch_types=scratch)
    def body(user_h, cur1_h, f1_o, idx_v, emb0_v, emb1_v, semg, sw0, sw1):
        wb = _wid() * SEEDS
        pltpu.sync_copy(cur1_h.at[:, pl.ds(wb, SEEDS)], idx_v)
        bufs = (emb0_v, emb1_v)
        sems = (sw0, sw1)
        wh = [None, None]
        for t in range(T):
            b = t % 2
            if wh[b] is not None:
                wh[b].wait()
            pltpu.async_copy(user_h.at[idx_v.at[t]], bufs[b], semg).wait()
            wh[b] = pltpu.async_copy(bufs[b], f1_o.at[t, pl.ds(wb, SEEDS)],
                                     sems[b])
        wh[0].wait()
        wh[1].wait()

    return body(user128, cur1)


def _sc_stage_c(user128, cur2):
    T = G * S1 * S0
    out_type = jax.ShapeDtypeStruct((T, B, EDP), jnp.float32)
    scratch = [
        pltpu.VMEM((T, SEEDS), jnp.int32),
        pltpu.VMEM((SEEDS, EDP), jnp.float32),
        pltpu.VMEM((SEEDS, EDP), jnp.float32),
        pltpu.SemaphoreType.DMA,
        pltpu.SemaphoreType.DMA,
        pltpu.SemaphoreType.DMA,
    ]

    @functools.partial(pl.kernel, out_type=out_type,
                       mesh=plsc.VectorSubcoreMesh(**_MESH),
                       compiler_params=_SC_LIN,
                       scratch_types=scratch)
    def body(user_h, cur2_h, f2_o, idx_v, emb0_v, emb1_v, semg, sw0, sw1):
        wb = _wid() * SEEDS
        pltpu.sync_copy(cur2_h.at[:, pl.ds(wb, SEEDS)], idx_v)
        bufs = (emb0_v, emb1_v)
        sems = (sw0, sw1)
        wh = [None, None]
        for t in range(T):
            b = t % 2
            if wh[b] is not None:
                wh[b].wait()
            pltpu.async_copy(user_h.at[idx_v.at[t]], bufs[b], semg).wait()
            wh[b] = pltpu.async_copy(bufs[b], f2_o.at[t, pl.ds(wb, SEEDS)],
                                     sems[b])
        wh[0].wait()
        wh[1].wait()

    return body(user128, cur2)


def _sel_body(ngroups):
    TSP = TS // 8

    def body(rows_r, off_r, out_r):
        dcol = lax.broadcasted_iota(jnp.int32, (TSP, DEG), 1)
        for t in range(ngroups):
            if ngroups == G * S0:
                row_idx = t // S0
                bias = (t // S0) * NPAD
            else:
                g = t // (S1 * S0)
                j = t % S0
                row_idx = g * S0 + j
                bias = g * NPAD
            rows = rows_r[row_idx]
            vals = []
            for j8 in range(8):
                sl = rows[:, j8 * DEG:(j8 + 1) * DEG]
                col = off_r[t, :, j8:j8 + 1]
                vals.append(
                    jnp.sum(jnp.where(dcol == col, sl, 0), axis=1))
            out_r[t] = jnp.stack(vals, axis=1) + bias
    return body


def _tc_select(rows_packed, off_packed, ngroups, nrows):
    TSP = TS // 8
    out = pl.pallas_call(
        _sel_body(ngroups),
        grid=(TGRID,),
        in_specs=[
            pl.BlockSpec((nrows, TSP, DEG * 8), lambda i: (0, i, 0)),
            pl.BlockSpec((ngroups, TSP, 8), lambda i: (0, i, 0)),
        ],
        out_specs=pl.BlockSpec((ngroups, TSP, 8), lambda i: (0, i, 0)),
        out_shape=jax.ShapeDtypeStruct((ngroups, B // 8, 8), jnp.int32),
    )(rows_packed, off_packed)
    return out.reshape(ngroups, B)


def _dense_body(f0_r, f1_r, f2_r, base_r, et_r,
                wx_r, bx_r, wn_r, bn_r, fcw_r, fcb_r,
                lng_r, lnb_r, wq_r, wk_r, wv_r, wo_r, refl_r, out_r):
    dot = functools.partial(jnp.dot, precision=jax.lax.Precision.HIGHEST,
                            preferred_element_type=jnp.float32)
    spec = []
    for g in range(G):
        wx0a = wx_r[g * L + 0, :HD, :]
        wx0b = wx_r[g * L + 0, HD:, :]
        wn0a = wn_r[g * L + 0, :HD, :]
        wn0b = wn_r[g * L + 0, HD:, :]
        bx0 = bx_r[g * L + 0]
        bn0 = bn_r[g * L + 0]
        wx1a = wx_r[g * L + 1, :HD, :]
        wx1b = wx_r[g * L + 1, HD:, :]
        wn1a = wn_r[g * L + 1, :HD, :]
        wn1b = wn_r[g * L + 1, HD:, :]
        bx1 = bx_r[g * L + 1]
        bn1 = bn_r[g * L + 1]

        f0 = f0_r[g]
        f1 = [f1_r[g * S0 + j] for j in range(S0)]
        h1a, h1b = [], []
        for j in range(S0):
            nb2 = f2_r[(g * S1 + 0) * S0 + j]
            for j2 in range(1, S1):
                nb2 = nb2 + f2_r[(g * S1 + j2) * S0 + j]
            nb2 = nb2 * (1.0 / S1)
            xa = dot(f1[j][:, :HD], wx0a) + dot(f1[j][:, HD:ED], wx0b) + bx0
            xb = dot(nb2[:, :HD], wn0a) + dot(nb2[:, HD:ED], wn0b) + bn0
            h1a.append(jnp.maximum(xa, 0.0))
            h1b.append(jnp.maximum(xb, 0.0))
        nb1 = (f1[0] + f1[1] + f1[2]) * (1.0 / S0)
        xa = dot(f0[:, :HD], wx0a) + dot(f0[:, HD:ED], wx0b) + bx0
        xb = dot(nb1[:, :HD], wn0a) + dot(nb1[:, HD:ED], wn0b) + bn0
        h0a = jnp.maximum(xa, 0.0)
        h0b = jnp.maximum(xb, 0.0)
        nha = (h1a[0] + h1a[1] + h1a[2]) * (1.0 / S0)
        nhb = (h1b[0] + h1b[1] + h1b[2]) * (1.0 / S0)
        ya = dot(h0a, wx1a) + dot(h0b, wx1b) + bx1
        yb = dot(nha, wn1a) + dot(nhb, wn1b) + bn1
        ha = jnp.maximum(ya, 0.0)
        hb = jnp.maximum(yb, 0.0)
        nrm2 = jnp.sum(ha * ha, axis=1, keepdims=True) + \
            jnp.sum(hb * hb, axis=1, keepdims=True)
        inv = 1.0 / jnp.maximum(jnp.sqrt(nrm2), 1e-12)
        fca = fcw_r[g, :HD, :]
        fcb_w = fcw_r[g, HD:, :]
        sg = (dot(ha, fca) + dot(hb, fcb_w)) * inv + fcb_r[g]
        spec.append(sg)

    lng = lng_r[0]
    lnb = lnb_r[0]
    q, k, v = [], [], []
    for g in range(G):
        s = spec[g]
        m = jnp.mean(s, axis=1, keepdims=True)
        var = jnp.mean((s - m) * (s - m), axis=1, keepdims=True)
        qn = (s - m) / jnp.sqrt(var + 1e-6) * lng + lnb
        q.append(dot(qn, wq_r[...]))
        k.append(dot(s, wk_r[...]))
        v.append(dot(s, wv_r[...]))
    scale = 1.0 / jnp.sqrt(jnp.float32(ED))
    o = []
    for qq in range(G):
        lg = [jnp.sum(q[qq] * k[kk], axis=1, keepdims=True) * scale
              for kk in range(G)]
        mx = jnp.maximum(jnp.maximum(lg[0], lg[1]), lg[2])
        e = [jnp.exp(x - mx) for x in lg]
        ssum = e[0] + e[1] + e[2]
        att = [x / ssum for x in e]
        ov = att[0] * v[0] + att[1] * v[1] + att[2] * v[2]
        o.append(dot(ov, wo_r[...]) + spec[qq])

    et = et_r[...]
    acc = jnp.zeros((TS, OD), jnp.float32)
    for g in range(G):
        pr = dot(o[g], refl_r[g])
        acc = acc + jnp.where(et == g, pr, 0.0)
    fin = base_r[:, :ID] + acc
    nrm = jnp.sqrt(jnp.sum(fin * fin, axis=1, keepdims=True))
    out_r[...] = fin / jnp.maximum(nrm, 1e-12)


def _dense_call(f0, f1, f2, baser, et2, agg_Wx, agg_bx, agg_Wn, agg_bn,
                fc_W, fc_b, ln_g, ln_b, wq, wk, wv, w_o, reflect):
    def blk(shape, im):
        return pl.BlockSpec(shape, im)

    def full(x):
        nd = x.ndim
        return pl.BlockSpec(x.shape, lambda t, nd=nd: (0,) * nd)

    in_specs = [
        blk((G, TS, EDP), lambda t: (0, t, 0)),
        blk((G * S0, TS, EDP), lambda t: (0, t, 0)),
        blk((G * S1 * S0, TS, EDP), lambda t: (0, t, 0)),
        blk((TS, IDP), lambda t: (t, 0)),
        blk((TS, 1), lambda t: (t, 0)),
        full(agg_Wx), full(agg_bx), full(agg_Wn), full(agg_bn),
        full(fc_W), full(fc_b), full(ln_g), full(ln_b),
        full(wq), full(wk), full(wv), full(w_o), full(reflect),
    ]
    return pl.pallas_call(
        _dense_body,
        grid=(TGRID,),
        in_specs=in_specs,
        out_specs=pl.BlockSpec((TS, OD), lambda t: (t, 0)),
        out_shape=jax.ShapeDtypeStruct((B, OD), jnp.float32),
    )(f0, f1, f2, baser, et2, agg_Wx, agg_bx, agg_Wn, agg_bn,
      fc_W, fc_b, ln_g, ln_b, wq, wk, wv, w_o, reflect)


def kernel(nodeids, edgetypes, nbrs, base_embed, user_embed, agg_Wx, agg_bx,
           agg_Wn, agg_bn, fc_W, fc_b, ln_g, ln_b, wq, wk, wv, w_o, reflect):
    skey = jax.random.key(42)
    off0, off1 = [], []
    for g in range(G):
        o0 = jax.random.randint(jax.random.fold_in(skey, g * 100 + 0),
                                (B, S0), 0, DEG, dtype=jnp.int32)
        o1 = jax.random.randint(jax.random.fold_in(skey, g * 100 + 1),
                                (B * S0, S1), 0, DEG, dtype=jnp.int32)
        off0.append(o0.T)
        off1.append(o1.reshape(B, S0, S1).transpose(2, 1, 0))
    off0 = jnp.stack(off0).reshape(G * S0, B // 8, 8)
    off1 = jnp.stack(off1).reshape(G * S1 * S0, B // 8, 8)

    nbrs_flat = nbrs.reshape(G * N, DEG)
    user128 = _tc_repack_user(jnp.transpose(user_embed, (0, 2, 1)))
    base256 = _tc_repack_base(jnp.transpose(base_embed, (1, 0)))

    rows0 = _sc_stage_a1(nodeids, nbrs_flat)
    f0, baser = _sc_stage_a2(nodeids, user128, base256)
    cur1 = _tc_select(rows0.reshape(G, B // 8, DEG * 8),
                      off0, G * S0, G)
    rows1 = _sc_stage_b1(nbrs_flat, cur1)
    f1 = _sc_stage_b2(user128, cur1)
    cur2 = _tc_select(rows1.reshape(G * S0, B // 8, DEG * 8),
                      off1, G * S1 * S0, G * S0)
    f2 = _sc_stage_c(user128, cur2)

    et2 = edgetypes.reshape(B, 1)
    lng2 = ln_g.reshape(1, ED)
    lnb2 = ln_b.reshape(1, ED)
    wx = agg_Wx.reshape(G * L, ED, HD)
    bx = agg_bx.reshape(G * L, HD)
    wn = agg_Wn.reshape(G * L, ED, HD)
    bn = agg_bn.reshape(G * L, HD)

    return _dense_call(f0, f1, f2, baser, et2, wx, bx, wn, bn,
                       fc_W, fc_b, lng2, lnb2, wq, wk, wv, w_o, reflect)


# speedup vs baseline: 2.0567x; 1.3325x over previous
import functools

import jax
import jax.numpy as jnp
from jax import lax
from jax.experimental import pallas as pl
from jax.experimental.pallas import tpu as pltpu
from jax.experimental.pallas import tpu_sc as plsc

N = 100000
DEG = 16
G = 3
L = 2
B = 4096
ED = 100
ID = 200
OD = 200
S0 = 3
S1 = 5

NC = 2
NS = 16
NW = NC * NS
SEEDS = B // NW

HD = ED // 2
EDP = 128
IDP = 256
RCH = 1024
NPAD = 100352
TS = 512
TGRID = B // TS

_MESH = dict(core_axis_name="c", subcore_axis_name="s")
_SC_LIN = pltpu.CompilerParams(use_tc_tiling_on_sc=False)
_SC_TILED = pltpu.CompilerParams(use_tc_tiling_on_sc=True)


def _tc_repack_user(u_t):
    def body(u_r, o_r):
        o_r[:, :ED] = lax.transpose(u_r[0], (1, 0))
        o_r[:, ED:] = jnp.zeros((RCH, EDP - ED), jnp.float32)

    return pl.pallas_call(
        body,
        grid=(G, NPAD // RCH),
        in_specs=[pl.BlockSpec((1, ED, RCH), lambda g, i: (g, 0, i))],
        out_specs=pl.BlockSpec((RCH, EDP),
                               lambda g, i: (g * (NPAD // RCH) + i, 0)),
        out_shape=jax.ShapeDtypeStruct((G * NPAD, EDP), jnp.float32),
    )(u_t)


def _tc_repack_base(b_t):
    def body(b_r, o_r):
        o_r[:, :ID] = lax.transpose(b_r[...], (1, 0))
        o_r[:, ID:] = jnp.zeros((RCH, IDP - ID), jnp.float32)

    return pl.pallas_call(
        body,
        grid=(NPAD // RCH,),
        in_specs=[pl.BlockSpec((ID, RCH), lambda i: (0, i))],
        out_specs=pl.BlockSpec((RCH, IDP), lambda i: (i, 0)),
        out_shape=jax.ShapeDtypeStruct((NPAD, IDP), jnp.float32),
    )(b_t)


def _wid():
    return lax.axis_index("s") * NC + lax.axis_index("c")


def _sc_stage_a1(nodeids, nbrs_flat):
    out_type = jax.ShapeDtypeStruct((G, B, DEG), jnp.int32)
    scratch = [
        pltpu.VMEM((SEEDS,), jnp.int32),
        pltpu.VMEM((SEEDS,), jnp.int32),
        pltpu.VMEM((SEEDS, DEG), jnp.int32),
        pltpu.SemaphoreType.DMA,
    ]

    @functools.partial(pl.kernel, out_type=out_type,
                       mesh=plsc.VectorSubcoreMesh(**_MESH),
                       compiler_params=_SC_LIN,
                       scratch_types=scratch)
    def body(nodeids_h, nbrs_h, rows0_o, seeds_v, idx_v, rows_v, sem):
        wb = _wid() * SEEDS
        pltpu.sync_copy(nodeids_h.at[pl.ds(wb, SEEDS)], seeds_v)
        for g in range(G):
            for c in range(SEEDS // 16):
                idx_v[pl.ds(c * 16, 16)] = seeds_v[pl.ds(c * 16, 16)] + g * N
            pltpu.async_copy(nbrs_h.at[idx_v], rows_v, sem).wait()
            pltpu.sync_copy(rows_v, rows0_o.at[g, pl.ds(wb, SEEDS)])

    return body(nodeids, nbrs_flat)


def _sc_stage_a2(nodeids, user128, base256):
    out_type = (
        jax.ShapeDtypeStruct((G, B, EDP), jnp.float32),
        jax.ShapeDtypeStruct((B, IDP), jnp.float32),
    )
    scratch = [
        pltpu.VMEM((SEEDS,), jnp.int32),
        pltpu.VMEM((SEEDS,), jnp.int32),
        pltpu.VMEM((SEEDS, EDP), jnp.float32),
        pltpu.VMEM((SEEDS, IDP), jnp.float32),
        pltpu.SemaphoreType.DMA,
    ]

    @functools.partial(pl.kernel, out_type=out_type,
                       mesh=plsc.VectorSubcoreMesh(**_MESH),
                       compiler_params=_SC_LIN,
                       scratch_types=scratch)
    def body(nodeids_h, user_h, base_h, f0_o, base_o,
             seeds_v, idx_v, emb_v, base_v, sem):
        wb = _wid() * SEEDS
        pltpu.sync_copy(nodeids_h.at[pl.ds(wb, SEEDS)], seeds_v)
        pltpu.async_copy(base_h.at[seeds_v], base_v, sem).wait()
        pltpu.sync_copy(base_v, base_o.at[pl.ds(wb, SEEDS)])
        for g in range(G):
            for c in range(SEEDS // 16):
                idx_v[pl.ds(c * 16, 16)] = seeds_v[pl.ds(c * 16, 16)] + g * NPAD
            pltpu.async_copy(user_h.at[idx_v], emb_v, sem).wait()
            pltpu.sync_copy(emb_v, f0_o.at[g, pl.ds(wb, SEEDS)])

    return body(nodeids, user128, base256)


def _sc_stage_b1(nbrs_flat, cur1):
    T = G * S0
    out_type = jax.ShapeDtypeStruct((T, B, DEG), jnp.int32)
    scratch = [
        pltpu.VMEM((SEEDS,), jnp.int32),
        pltpu.VMEM((SEEDS, DEG), jnp.int32),
        pltpu.SemaphoreType.DMA,
    ]

    @functools.partial(pl.kernel, out_type=out_type,
                       mesh=plsc.VectorSubcoreMesh(**_MESH),
                       compiler_params=_SC_LIN,
                       scratch_types=scratch)
    def body(nbrs_h, cur1_h, rows1_o, idx_v, rows_v, sem):
        wb = _wid() * SEEDS
        for t in range(T):
            pltpu.sync_copy(cur1_h.at[t, pl.ds(wb, SEEDS)], idx_v)
            g = t // S0
            if g:
                for c in range(SEEDS // 16):
                    idx_v[pl.ds(c * 16, 16)] = (idx_v[pl.ds(c * 16, 16)]
                                                - g * (NPAD - N))
            pltpu.async_copy(nbrs_h.at[idx_v], rows_v, sem).wait()
            pltpu.sync_copy(rows_v, rows1_o.at[t, pl.ds(wb, SEEDS)])

    return body(nbrs_flat, cur1)


def _sc_stage_b2(user128, cur1):
    T = G * S0
    out_type = jax.ShapeDtypeStruct((T, B, EDP), jnp.float32)
    scratch = [
        pltpu.VMEM((T, SEEDS), jnp.int32),
        pltpu.VMEM((SEEDS, EDP), jnp.float32),
        pltpu.VMEM((SEEDS, EDP), jnp.float32),
        pltpu.SemaphoreType.DMA,
        pltpu.SemaphoreType.DMA,
        pltpu.SemaphoreType.DMA,
    ]

    @functools.partial(pl.kernel, out_type=out_type,
                       mesh=plsc.VectorSubcoreMesh(**_MESH),
                       compiler_params=_SC_LIN,
                       scratch_types=scratch)
    def body(user_h, cur1_h, f1_o, idx_v, emb0_v, emb1_v, semg, sw0, sw1):
        wb = _wid() * SEEDS
        pltpu.sync_copy(cur1_h.at[:, pl.ds(wb, SEEDS)], idx_v)
        bufs = (emb0_v, emb1_v)
        sems = (sw0, sw1)
        wh = [None, None]
        for t in range(T):
            b = t % 2
            if wh[b] is not None:
                wh[b].wait()
            pltpu.async_copy(user_h.at[idx_v.at[t]], bufs[b], semg).wait()
            wh[b] = pltpu.async_copy(bufs[b], f1_o.at[t, pl.ds(wb, SEEDS)],
                                     sems[b])
        wh[0].wait()
        wh[1].wait()

    return body(user128, cur1)


def _sc_stage_c(user128, cur2):
    T = G * S1 * S0
    out_type = jax.ShapeDtypeStruct((T, B, EDP), jnp.float32)
    scratch = [
        pltpu.VMEM((T, SEEDS), jnp.int32),
        pltpu.VMEM((SEEDS, EDP), jnp.float32),
        pltpu.VMEM((SEEDS, EDP), jnp.float32),
        pltpu.SemaphoreType.DMA,
        pltpu.SemaphoreType.DMA,
        pltpu.SemaphoreType.DMA,
    ]

    @functools.partial(pl.kernel, out_type=out_type,
                       mesh=plsc.VectorSubcoreMesh(**_MESH),
                       compiler_params=_SC_LIN,
                       scratch_types=scratch)
    def body(user_h, cur2_h, f2_o, idx_v, emb0_v, emb1_v, semg, sw0, sw1):
        wb = _wid() * SEEDS
        pltpu.sync_copy(cur2_h.at[:, pl.ds(wb, SEEDS)], idx_v)
        bufs = (emb0_v, emb1_v)
        sems = (sw0, sw1)
        wh = [None, None]
        gh = [None, None]
        gh[0] = pltpu.async_copy(user_h.at[idx_v.at[0]], bufs[0], semg)
        for t in range(T):
            b = t % 2
            nb = (t + 1) % 2
            if t + 1 < T:
                if wh[nb] is not None:
                    wh[nb].wait()
                gh[nb] = pltpu.async_copy(user_h.at[idx_v.at[t + 1]],
                                          bufs[nb], semg)
            gh[b].wait()
            wh[b] = pltpu.async_copy(bufs[b], f2_o.at[t, pl.ds(wb, SEEDS)],
                                     sems[b])
        wh[0].wait()
        wh[1].wait()

    return body(user128, cur2)


def _sel_body(ngroups):
    TSP = TS // 8

    def body(rows_r, mask_r, out_r):
        grp = (lax.broadcasted_iota(jnp.int32, (DEG * 8, 8), 0) // DEG ==
               lax.broadcasted_iota(jnp.int32, (DEG * 8, 8), 1)
               ).astype(jnp.float32)
        for t in range(ngroups):
            if ngroups == G * S0:
                row_idx = t // S0
                bias = (t // S0) * NPAD
            else:
                g = t // (S1 * S0)
                j = t % S0
                row_idx = g * S0 + j
                bias = g * NPAD
            picked = rows_r[row_idx].astype(jnp.float32) * mask_r[t]
            val = jnp.dot(picked, grp,
                          precision=jax.lax.Precision.HIGHEST,
                          preferred_element_type=jnp.float32)
            out_r[t] = val.astype(jnp.int32) + bias
    return body


def _tc_select(rows_packed, mask, ngroups, nrows):
    TSP = TS // 8
    out = pl.pallas_call(
        _sel_body(ngroups),
        grid=(TGRID,),
        in_specs=[
            pl.BlockSpec((nrows, TSP, DEG * 8), lambda i: (0, i, 0)),
            pl.BlockSpec((ngroups, TSP, DEG * 8), lambda i: (0, i, 0)),
        ],
        out_specs=pl.BlockSpec((ngroups, TSP, 8), lambda i: (0, i, 0)),
        out_shape=jax.ShapeDtypeStruct((ngroups, B // 8, 8), jnp.int32),
    )(rows_packed, mask)
    return out.reshape(ngroups, B)


def _dense_body(f0_r, f1_r, f2_r, base_r, et_r,
                wx_r, bx_r, wn_r, bn_r, fcw_r, fcb_r,
                lng_r, lnb_r, wq_r, wk_r, wv_r, wo_r, refl_r, out_r):
    dot = functools.partial(jnp.dot, precision=jax.lax.Precision.HIGHEST,
                            preferred_element_type=jnp.float32)
    spec = []
    for g in range(G):
        wx0a = wx_r[g * L + 0, :HD, :]
        wx0b = wx_r[g * L + 0, HD:, :]
        wn0a = wn_r[g * L + 0, :HD, :]
        wn0b = wn_r[g * L + 0, HD:, :]
        bx0 = bx_r[g * L + 0]
        bn0 = bn_r[g * L + 0]
        wx1a = wx_r[g * L + 1, :HD, :]
        wx1b = wx_r[g * L + 1, HD:, :]
        wn1a = wn_r[g * L + 1, :HD, :]
        wn1b = wn_r[g * L + 1, HD:, :]
        bx1 = bx_r[g * L + 1]
        bn1 = bn_r[g * L + 1]

        f0 = f0_r[g]
        f1 = [f1_r[g * S0 + j] for j in range(S0)]
        h1a, h1b = [], []
        for j in range(S0):
            nb2 = f2_r[(g * S1 + 0) * S0 + j]
            for j2 in range(1, S1):
                nb2 = nb2 + f2_r[(g * S1 + j2) * S0 + j]
            nb2 = nb2 * (1.0 / S1)
            xa = dot(f1[j][:, :HD], wx0a) + dot(f1[j][:, HD:ED], wx0b) + bx0
            xb = dot(nb2[:, :HD], wn0a) + dot(nb2[:, HD:ED], wn0b) + bn0
            h1a.append(jnp.maximum(xa, 0.0))
            h1b.append(jnp.maximum(xb, 0.0))
        nb1 = (f1[0] + f1[1] + f1[2]) * (1.0 / S0)
        xa = dot(f0[:, :HD], wx0a) + dot(f0[:, HD:ED], wx0b) + bx0
        xb = dot(nb1[:, :HD], wn0a) + dot(nb1[:, HD:ED], wn0b) + bn0
        h0a = jnp.maximum(xa, 0.0)
        h0b = jnp.maximum(xb, 0.0)
        nha = (h1a[0] + h1a[1] + h1a[2]) * (1.0 / S0)
        nhb = (h1b[0] + h1b[1] + h1b[2]) * (1.0 / S0)
        ya = dot(h0a, wx1a) + dot(h0b, wx1b) + bx1
        yb = dot(nha, wn1a) + dot(nhb, wn1b) + bn1
        ha = jnp.maximum(ya, 0.0)
        hb = jnp.maximum(yb, 0.0)
        nrm2 = jnp.sum(ha * ha, axis=1, keepdims=True) + \
            jnp.sum(hb * hb, axis=1, keepdims=True)
        inv = 1.0 / jnp.maximum(jnp.sqrt(nrm2), 1e-12)
        fca = fcw_r[g, :HD, :]
        fcb_w = fcw_r[g, HD:, :]
        sg = (dot(ha, fca) + dot(hb, fcb_w)) * inv + fcb_r[g]
        spec.append(sg)

    lng = lng_r[0]
    lnb = lnb_r[0]
    q, k, v = [], [], []
    for g in range(G):
        s = spec[g]
        m = jnp.mean(s, axis=1, keepdims=True)
        var = jnp.mean((s - m) * (s - m), axis=1, keepdims=True)
        qn = (s - m) / jnp.sqrt(var + 1e-6) * lng + lnb
        q.append(dot(qn, wq_r[...]))
        k.append(dot(s, wk_r[...]))
        v.append(dot(s, wv_r[...]))
    scale = 1.0 / jnp.sqrt(jnp.float32(ED))
    o = []
    for qq in range(G):
        lg = [jnp.sum(q[qq] * k[kk], axis=1, keepdims=True) * scale
              for kk in range(G)]
        mx = jnp.maximum(jnp.maximum(lg[0], lg[1]), lg[2])
        e = [jnp.exp(x - mx) for x in lg]
        ssum = e[0] + e[1] + e[2]
        att = [x / ssum for x in e]
        ov = att[0] * v[0] + att[1] * v[1] + att[2] * v[2]
        o.append(dot(ov, wo_r[...]) + spec[qq])

    et = et_r[...]
    acc = jnp.zeros((TS, OD), jnp.float32)
    for g in range(G):
        pr = dot(o[g], refl_r[g])
        acc = acc + jnp.where(et == g, pr, 0.0)
    fin = base_r[:, :ID] + acc
    nrm = jnp.sqrt(jnp.sum(fin * fin, axis=1, keepdims=True))
    out_r[...] = fin / jnp.maximum(nrm, 1e-12)


def _dense_call(f0, f1, f2, baser, et2, agg_Wx, agg_bx, agg_Wn, agg_bn,
                fc_W, fc_b, ln_g, ln_b, wq, wk, wv, w_o, reflect):
    def blk(shape, im):
        return pl.BlockSpec(shape, im)

    def full(x):
        nd = x.ndim
        return pl.BlockSpec(x.shape, lambda t, nd=nd: (0,) * nd)

    in_specs = [
        blk((G, TS, EDP), lambda t: (0, t, 0)),
        blk((G * S0, TS, EDP), lambda t: (0, t, 0)),
        blk((G * S1 * S0, TS, EDP), lambda t: (0, t, 0)),
        blk((TS, IDP), lambda t: (t, 0)),
        blk((TS, 1), lambda t: (t, 0)),
        full(agg_Wx), full(agg_bx), full(agg_Wn), full(agg_bn),
        full(fc_W), full(fc_b), full(ln_g), full(ln_b),
        full(wq), full(wk), full(wv), full(w_o), full(reflect),
    ]
    return pl.pallas_call(
        _dense_body,
        grid=(TGRID,),
        in_specs=in_specs,
        out_specs=pl.BlockSpec((TS, OD), lambda t: (t, 0)),
        out_shape=jax.ShapeDtypeStruct((B, OD), jnp.float32),
    )(f0, f1, f2, baser, et2, agg_Wx, agg_bx, agg_Wn, agg_bn,
      fc_W, fc_b, ln_g, ln_b, wq, wk, wv, w_o, reflect)


def kernel(nodeids, edgetypes, nbrs, base_embed, user_embed, agg_Wx, agg_bx,
           agg_Wn, agg_bn, fc_W, fc_b, ln_g, ln_b, wq, wk, wv, w_o, reflect):
    skey = jax.random.key(42)
    off0, off1 = [], []
    for g in range(G):
        o0 = jax.random.randint(jax.random.fold_in(skey, g * 100 + 0),
                                (B, S0), 0, DEG, dtype=jnp.int32)
        o1 = jax.random.randint(jax.random.fold_in(skey, g * 100 + 1),
                                (B * S0, S1), 0, DEG, dtype=jnp.int32)
        off0.append(o0.T)
        off1.append(o1.reshape(B, S0, S1).transpose(2, 1, 0))
    lane = jnp.arange(DEG, dtype=jnp.int32)
    off0 = jnp.stack(off0).reshape(G * S0, B // 8, 8)
    off1 = jnp.stack(off1).reshape(G * S1 * S0, B // 8, 8)
    mask0 = (off0[..., None] == lane).astype(jnp.float32)
    mask0 = mask0.reshape(G * S0, B // 8, DEG * 8)
    mask1 = (off1[..., None] == lane).astype(jnp.float32)
    mask1 = mask1.reshape(G * S1 * S0, B // 8, DEG * 8)

    nbrs_flat = nbrs.reshape(G * N, DEG)
    user128 = _tc_repack_user(jnp.transpose(user_embed, (0, 2, 1)))
    base256 = _tc_repack_base(jnp.transpose(base_embed, (1, 0)))

    rows0 = _sc_stage_a1(nodeids, nbrs_flat)
    f0, baser = _sc_stage_a2(nodeids, user128, base256)
    cur1 = _tc_select(rows0.reshape(G, B // 8, DEG * 8),
                      mask0, G * S0, G)
    rows1 = _sc_stage_b1(nbrs_flat, cur1)
    f1 = _sc_stage_b2(user128, cur1)
    cur2 = _tc_select(rows1.reshape(G * S0, B // 8, DEG * 8),
                      mask1, G * S1 * S0, G * S0)
    f2 = _sc_stage_c(user128, cur2)

    et2 = edgetypes.reshape(B, 1)
    lng2 = ln_g.reshape(1, ED)
    lnb2 = ln_b.reshape(1, ED)
    wx = agg_Wx.reshape(G * L, ED, HD)
    bx = agg_bx.reshape(G * L, HD)
    wn = agg_Wn.reshape(G * L, ED, HD)
    bn = agg_bn.reshape(G * L, HD)

    return _dense_call(f0, f1, f2, baser, et2, wx, bx, wn, bn,
                       fc_W, fc_b, lng2, lnb2, wq, wk, wv, w_o, reflect)


# speedup vs baseline: 2.3518x; 1.1435x over previous
import functools

import jax
import jax.numpy as jnp
from jax import lax
from jax.experimental import pallas as pl
from jax.experimental.pallas import tpu as pltpu
from jax.experimental.pallas import tpu_sc as plsc

N = 100000
DEG = 16
G = 3
L = 2
B = 4096
ED = 100
ID = 200
OD = 200
S0 = 3
S1 = 5

NC = 2
NS = 16
NW = NC * NS
SEEDS = B // NW

HD = ED // 2
EDP = 128
IDP = 256
RCH = 1024
NPAD = 100352
TS = 512
TGRID = B // TS

_MESH = dict(core_axis_name="c", subcore_axis_name="s")
_SC_LIN = pltpu.CompilerParams(use_tc_tiling_on_sc=False)
_SC_TILED = pltpu.CompilerParams(use_tc_tiling_on_sc=True)


def _tc_repack_user(u_t):
    def body(u_r, o_r):
        o_r[:, :ED] = lax.transpose(u_r[0], (1, 0))
        o_r[:, ED:] = jnp.zeros((RCH, EDP - ED), jnp.float32)

    return pl.pallas_call(
        body,
        grid=(G, NPAD // RCH),
        in_specs=[pl.BlockSpec((1, ED, RCH), lambda g, i: (g, 0, i))],
        out_specs=pl.BlockSpec((RCH, EDP),
                               lambda g, i: (g * (NPAD // RCH) + i, 0)),
        out_shape=jax.ShapeDtypeStruct((G * NPAD, EDP), jnp.float32),
    )(u_t)


def _tc_repack_base(b_t):
    def body(b_r, o_r):
        o_r[:, :ID] = lax.transpose(b_r[...], (1, 0))
        o_r[:, ID:] = jnp.zeros((RCH, IDP - ID), jnp.float32)

    return pl.pallas_call(
        body,
        grid=(NPAD // RCH,),
        in_specs=[pl.BlockSpec((ID, RCH), lambda i: (0, i))],
        out_specs=pl.BlockSpec((RCH, IDP), lambda i: (i, 0)),
        out_shape=jax.ShapeDtypeStruct((NPAD, IDP), jnp.float32),
    )(b_t)


def _wid():
    return lax.axis_index("s") * NC + lax.axis_index("c")


def _sc_stage_a1(nodeids, nbrs_flat):
    out_type = jax.ShapeDtypeStruct((G, B, DEG), jnp.int32)
    scratch = [
        pltpu.VMEM((SEEDS,), jnp.int32),
        pltpu.VMEM((SEEDS,), jnp.int32),
        pltpu.VMEM((SEEDS, DEG), jnp.int32),
        pltpu.SemaphoreType.DMA,
    ]

    @functools.partial(pl.kernel, out_type=out_type,
                       mesh=plsc.VectorSubcoreMesh(**_MESH),
                       compiler_params=_SC_LIN,
                       scratch_types=scratch)
    def body(nodeids_h, nbrs_h, rows0_o, seeds_v, idx_v, rows_v, sem):
        wb = _wid() * SEEDS
        pltpu.sync_copy(nodeids_h.at[pl.ds(wb, SEEDS)], seeds_v)
        for g in range(G):
            for c in range(SEEDS // 16):
                idx_v[pl.ds(c * 16, 16)] = seeds_v[pl.ds(c * 16, 16)] + g * N
            pltpu.async_copy(nbrs_h.at[idx_v], rows_v, sem).wait()
            pltpu.sync_copy(rows_v, rows0_o.at[g, pl.ds(wb, SEEDS)])

    return body(nodeids, nbrs_flat)


def _sc_stage_a2(nodeids, user128, base256):
    out_type = (
        jax.ShapeDtypeStruct((G, B, EDP), jnp.float32),
        jax.ShapeDtypeStruct((B, IDP), jnp.float32),
    )
    scratch = [
        pltpu.VMEM((SEEDS,), jnp.int32),
        pltpu.VMEM((SEEDS,), jnp.int32),
        pltpu.VMEM((SEEDS, EDP), jnp.float32),
        pltpu.VMEM((SEEDS, IDP), jnp.float32),
        pltpu.SemaphoreType.DMA,
    ]

    @functools.partial(pl.kernel, out_type=out_type,
                       mesh=plsc.VectorSubcoreMesh(**_MESH),
                       compiler_params=_SC_LIN,
                       scratch_types=scratch)
    def body(nodeids_h, user_h, base_h, f0_o, base_o,
             seeds_v, idx_v, emb_v, base_v, sem):
        wb = _wid() * SEEDS
        pltpu.sync_copy(nodeids_h.at[pl.ds(wb, SEEDS)], seeds_v)
        pltpu.async_copy(base_h.at[seeds_v], base_v, sem).wait()
        pltpu.sync_copy(base_v, base_o.at[pl.ds(wb, SEEDS)])
        for g in range(G):
            for c in range(SEEDS // 16):
                idx_v[pl.ds(c * 16, 16)] = seeds_v[pl.ds(c * 16, 16)] + g * NPAD
            pltpu.async_copy(user_h.at[idx_v], emb_v, sem).wait()
            pltpu.sync_copy(emb_v, f0_o.at[g, pl.ds(wb, SEEDS)])

    return body(nodeids, user128, base256)


def _sc_stage_b1(nbrs_flat, cur1):
    T = G * S0
    out_type = jax.ShapeDtypeStruct((T, B, DEG), jnp.int32)
    scratch = [
        pltpu.VMEM((SEEDS,), jnp.int32),
        pltpu.VMEM((SEEDS, DEG), jnp.int32),
        pltpu.SemaphoreType.DMA,
    ]

    @functools.partial(pl.kernel, out_type=out_type,
                       mesh=plsc.VectorSubcoreMesh(**_MESH),
                       compiler_params=_SC_LIN,
                       scratch_types=scratch)
    def body(nbrs_h, cur1_h, rows1_o, idx_v, rows_v, sem):
        wb = _wid() * SEEDS
        for t in range(T):
            pltpu.sync_copy(cur1_h.at[t, pl.ds(wb, SEEDS)], idx_v)
            g = t // S0
            if g:
                for c in range(SEEDS // 16):
                    idx_v[pl.ds(c * 16, 16)] = (idx_v[pl.ds(c * 16, 16)]
                                                - g * (NPAD - N))
            pltpu.async_copy(nbrs_h.at[idx_v], rows_v, sem).wait()
            pltpu.sync_copy(rows_v, rows1_o.at[t, pl.ds(wb, SEEDS)])

    return body(nbrs_flat, cur1)


def _sc_stage_b2(user128, cur1):
    T = G * S0
    out_type = jax.ShapeDtypeStruct((T, B, EDP), jnp.float32)
    scratch = [
        pltpu.VMEM((T, SEEDS), jnp.int32),
        pltpu.VMEM((SEEDS, EDP), jnp.float32),
        pltpu.VMEM((SEEDS, EDP), jnp.float32),
        pltpu.SemaphoreType.DMA,
        pltpu.SemaphoreType.DMA,
        pltpu.SemaphoreType.DMA,
    ]

    @functools.partial(pl.kernel, out_type=out_type,
                       mesh=plsc.VectorSubcoreMesh(**_MESH),
                       compiler_params=_SC_LIN,
                       scratch_types=scratch)
    def body(user_h, cur1_h, f1_o, idx_v, emb0_v, emb1_v, semg, sw0, sw1):
        wb = _wid() * SEEDS
        pltpu.sync_copy(cur1_h.at[:, pl.ds(wb, SEEDS)], idx_v)
        bufs = (emb0_v, emb1_v)
        sems = (sw0, sw1)
        wh = [None, None]
        for t in range(T):
            b = t % 2
            if wh[b] is not None:
                wh[b].wait()
            pltpu.async_copy(user_h.at[idx_v.at[t]], bufs[b], semg).wait()
            wh[b] = pltpu.async_copy(bufs[b], f1_o.at[t, pl.ds(wb, SEEDS)],
                                     sems[b])
        wh[0].wait()
        wh[1].wait()

    return body(user128, cur1)


def _sc_stage_c(user128, cur2):
    T = G * S1 * S0
    out_type = jax.ShapeDtypeStruct((T, B, EDP), jnp.float32)
    scratch = [
        pltpu.VMEM((T, SEEDS), jnp.int32),
        pltpu.VMEM((SEEDS, EDP), jnp.float32),
        pltpu.VMEM((SEEDS, EDP), jnp.float32),
        pltpu.SemaphoreType.DMA,
        pltpu.SemaphoreType.DMA,
        pltpu.SemaphoreType.DMA,
    ]

    @functools.partial(pl.kernel, out_type=out_type,
                       mesh=plsc.VectorSubcoreMesh(**_MESH),
                       compiler_params=_SC_LIN,
                       scratch_types=scratch)
    def body(user_h, cur2_h, f2_o, idx_v, emb0_v, emb1_v, semg, sw0, sw1):
        wb = _wid() * SEEDS
        pltpu.sync_copy(cur2_h.at[:, pl.ds(wb, SEEDS)], idx_v)
        bufs = (emb0_v, emb1_v)
        sems = (sw0, sw1)
        wh = [None, None]
        gh = [None, None]
        gh[0] = pltpu.async_copy(user_h.at[idx_v.at[0]], bufs[0], semg)
        for t in range(T):
            b = t % 2
            nb = (t + 1) % 2
            if t + 1 < T:
                if wh[nb] is not None:
                    wh[nb].wait()
                gh[nb] = pltpu.async_copy(user_h.at[idx_v.at[t + 1]],
                                          bufs[nb], semg)
            gh[b].wait()
            wh[b] = pltpu.async_copy(bufs[b], f2_o.at[t, pl.ds(wb, SEEDS)],
                                     sems[b])
        wh[0].wait()
        wh[1].wait()

    return body(user128, cur2)


def _sel_body(ngroups):
    TSP = TS // 8

    def body(rows_r, mask_r, out_r):
        grp = (lax.broadcasted_iota(jnp.int32, (DEG * 8, 8), 0) // DEG ==
               lax.broadcasted_iota(jnp.int32, (DEG * 8, 8), 1)
               ).astype(jnp.float32)
        for t in range(ngroups):
            if ngroups == G * S0:
                row_idx = t // S0
                bias = (t // S0) * NPAD
            else:
                g = t // (S1 * S0)
                j = t % S0
                row_idx = g * S0 + j
                bias = g * NPAD
            picked = rows_r[row_idx].astype(jnp.float32) * mask_r[t]
            val = jnp.dot(picked, grp,
                          precision=jax.lax.Precision.HIGHEST,
                          preferred_element_type=jnp.float32)
            out_r[t] = val.astype(jnp.int32) + bias
    return body


def _tc_select(rows_packed, mask, ngroups, nrows):
    TSP = TS // 8
    out = pl.pallas_call(
        _sel_body(ngroups),
        grid=(TGRID,),
        in_specs=[
            pl.BlockSpec((nrows, TSP, DEG * 8), lambda i: (0, i, 0)),
            pl.BlockSpec((ngroups, TSP, DEG * 8), lambda i: (0, i, 0)),
        ],
        out_specs=pl.BlockSpec((ngroups, TSP, 8), lambda i: (0, i, 0)),
        out_shape=jax.ShapeDtypeStruct((ngroups, B // 8, 8), jnp.int32),
    )(rows_packed, mask)
    return out.reshape(ngroups, B)


def _dense_body(f0_r, f1_r, f2_r, base_r, et_r,
                wx_r, bx_r, wn_r, bn_r, fcw_r, fcb_r,
                lng_r, lnb_r, wq_r, wk_r, wv_r, wo_r, refl_r, out_r):
    dot = functools.partial(jnp.dot, preferred_element_type=jnp.float32)
    spec = []
    for g in range(G):
        wx0a = wx_r[g * L + 0, :HD, :]
        wx0b = wx_r[g * L + 0, HD:, :]
        wn0a = wn_r[g * L + 0, :HD, :]
        wn0b = wn_r[g * L + 0, HD:, :]
        bx0 = bx_r[g * L + 0]
        bn0 = bn_r[g * L + 0]
        wx1a = wx_r[g * L + 1, :HD, :]
        wx1b = wx_r[g * L + 1, HD:, :]
        wn1a = wn_r[g * L + 1, :HD, :]
        wn1b = wn_r[g * L + 1, HD:, :]
        bx1 = bx_r[g * L + 1]
        bn1 = bn_r[g * L + 1]

        f0 = f0_r[g]
        f1 = [f1_r[g * S0 + j] for j in range(S0)]
        h1a, h1b = [], []
        for j in range(S0):
            nb2 = f2_r[(g * S1 + 0) * S0 + j]
            for j2 in range(1, S1):
                nb2 = nb2 + f2_r[(g * S1 + j2) * S0 + j]
            nb2 = nb2 * (1.0 / S1)
            xa = dot(f1[j][:, :HD], wx0a) + dot(f1[j][:, HD:ED], wx0b) + bx0
            xb = dot(nb2[:, :HD], wn0a) + dot(nb2[:, HD:ED], wn0b) + bn0
            h1a.append(jnp.maximum(xa, 0.0))
            h1b.append(jnp.maximum(xb, 0.0))
        nb1 = (f1[0] + f1[1] + f1[2]) * (1.0 / S0)
        xa = dot(f0[:, :HD], wx0a) + dot(f0[:, HD:ED], wx0b) + bx0
        xb = dot(nb1[:, :HD], wn0a) + dot(nb1[:, HD:ED], wn0b) + bn0
        h0a = jnp.maximum(xa, 0.0)
        h0b = jnp.maximum(xb, 0.0)
        nha = (h1a[0] + h1a[1] + h1a[2]) * (1.0 / S0)
        nhb = (h1b[0] + h1b[1] + h1b[2]) * (1.0 / S0)
        ya = dot(h0a, wx1a) + dot(h0b, wx1b) + bx1
        yb = dot(nha, wn1a) + dot(nhb, wn1b) + bn1
        ha = jnp.maximum(ya, 0.0)
        hb = jnp.maximum(yb, 0.0)
        nrm2 = jnp.sum(ha * ha, axis=1, keepdims=True) + \
            jnp.sum(hb * hb, axis=1, keepdims=True)
        inv = 1.0 / jnp.maximum(jnp.sqrt(nrm2), 1e-12)
        fca = fcw_r[g, :HD, :]
        fcb_w = fcw_r[g, HD:, :]
        sg = (dot(ha, fca) + dot(hb, fcb_w)) * inv + fcb_r[g]
        spec.append(sg)

    lng = lng_r[0]
    lnb = lnb_r[0]
    q, k, v = [], [], []
    for g in range(G):
        s = spec[g]
        m = jnp.mean(s, axis=1, keepdims=True)
        var = jnp.mean((s - m) * (s - m), axis=1, keepdims=True)
        qn = (s - m) / jnp.sqrt(var + 1e-6) * lng + lnb
        q.append(dot(qn, wq_r[...]))
        k.append(dot(s, wk_r[...]))
        v.append(dot(s, wv_r[...]))
    scale = 1.0 / jnp.sqrt(jnp.float32(ED))
    o = []
    for qq in range(G):
        lg = [jnp.sum(q[qq] * k[kk], axis=1, keepdims=True) * scale
              for kk in range(G)]
        mx = jnp.maximum(jnp.maximum(lg[0], lg[1]), lg[2])
        e = [jnp.exp(x - mx) for x in lg]
        ssum = e[0] + e[1] + e[2]
        att = [x / ssum for x in e]
        ov = att[0] * v[0] + att[1] * v[1] + att[2] * v[2]
        o.append(dot(ov, wo_r[...]) + spec[qq])

    et = et_r[...]
    acc = jnp.zeros((TS, OD), jnp.float32)
    for g in range(G):
        pr = dot(o[g], refl_r[g])
        acc = acc + jnp.where(et == g, pr, 0.0)
    fin = base_r[:, :ID] + acc
    nrm = jnp.sqrt(jnp.sum(fin * fin, axis=1, keepdims=True))
    out_r[...] = fin / jnp.maximum(nrm, 1e-12)


def _dense_call(f0, f1, f2, baser, et2, agg_Wx, agg_bx, agg_Wn, agg_bn,
                fc_W, fc_b, ln_g, ln_b, wq, wk, wv, w_o, reflect):
    def blk(shape, im):
        return pl.BlockSpec(shape, im)

    def full(x):
        nd = x.ndim
        return pl.BlockSpec(x.shape, lambda t, nd=nd: (0,) * nd)

    in_specs = [
        blk((G, TS, EDP), lambda t: (0, t, 0)),
        blk((G * S0, TS, EDP), lambda t: (0, t, 0)),
        blk((G * S1 * S0, TS, EDP), lambda t: (0, t, 0)),
        blk((TS, IDP), lambda t: (t, 0)),
        blk((TS, 1), lambda t: (t, 0)),
        full(agg_Wx), full(agg_bx), full(agg_Wn), full(agg_bn),
        full(fc_W), full(fc_b), full(ln_g), full(ln_b),
        full(wq), full(wk), full(wv), full(w_o), full(reflect),
    ]
    return pl.pallas_call(
        _dense_body,
        grid=(TGRID,),
        in_specs=in_specs,
        out_specs=pl.BlockSpec((TS, OD), lambda t: (t, 0)),
        out_shape=jax.ShapeDtypeStruct((B, OD), jnp.float32),
    )(f0, f1, f2, baser, et2, agg_Wx, agg_bx, agg_Wn, agg_bn,
      fc_W, fc_b, ln_g, ln_b, wq, wk, wv, w_o, reflect)


def kernel(nodeids, edgetypes, nbrs, base_embed, user_embed, agg_Wx, agg_bx,
           agg_Wn, agg_bn, fc_W, fc_b, ln_g, ln_b, wq, wk, wv, w_o, reflect):
    skey = jax.random.key(42)
    off0, off1 = [], []
    for g in range(G):
        o0 = jax.random.randint(jax.random.fold_in(skey, g * 100 + 0),
                                (B, S0), 0, DEG, dtype=jnp.int32)
        o1 = jax.random.randint(jax.random.fold_in(skey, g * 100 + 1),
                                (B * S0, S1), 0, DEG, dtype=jnp.int32)
        off0.append(o0.T)
        off1.append(o1.reshape(B, S0, S1).transpose(2, 1, 0))
    lane = jnp.arange(DEG, dtype=jnp.int32)
    off0 = jnp.stack(off0).reshape(G * S0, B // 8, 8)
    off1 = jnp.stack(off1).reshape(G * S1 * S0, B // 8, 8)
    mask0 = (off0[..., None] == lane).astype(jnp.float32)
    mask0 = mask0.reshape(G * S0, B // 8, DEG * 8)
    mask1 = (off1[..., None] == lane).astype(jnp.float32)
    mask1 = mask1.reshape(G * S1 * S0, B // 8, DEG * 8)

    nbrs_flat = nbrs.reshape(G * N, DEG)
    user128 = _tc_repack_user(jnp.transpose(user_embed, (0, 2, 1)))
    base256 = _tc_repack_base(jnp.transpose(base_embed, (1, 0)))

    rows0 = _sc_stage_a1(nodeids, nbrs_flat)
    f0, baser = _sc_stage_a2(nodeids, user128, base256)
    cur1 = _tc_select(rows0.reshape(G, B // 8, DEG * 8),
                      mask0, G * S0, G)
    rows1 = _sc_stage_b1(nbrs_flat, cur1)
    f1 = _sc_stage_b2(user128, cur1)
    cur2 = _tc_select(rows1.reshape(G * S0, B // 8, DEG * 8),
                      mask1, G * S1 * S0, G * S0)
    f2 = _sc_stage_c(user128, cur2)

    et2 = edgetypes.reshape(B, 1)
    lng2 = ln_g.reshape(1, ED)
    lnb2 = ln_b.reshape(1, ED)
    wx = agg_Wx.reshape(G * L, ED, HD)
    bx = agg_bx.reshape(G * L, HD)
    wn = agg_Wn.reshape(G * L, ED, HD)
    bn = agg_bn.reshape(G * L, HD)

    return _dense_call(f0, f1, f2, baser, et2, wx, bx, wn, bn,
                       fc_W, fc_b, lng2, lnb2, wq, wk, wv, w_o, reflect)


# speedup vs baseline: 2.8091x; 1.1945x over previous
import functools

import jax
import jax.numpy as jnp
from jax import lax
from jax.experimental import pallas as pl
from jax.experimental.pallas import tpu as pltpu
from jax.experimental.pallas import tpu_sc as plsc

N = 100000
DEG = 16
G = 3
L = 2
B = 4096
ED = 100
ID = 200
OD = 200
S0 = 3
S1 = 5

NC = 2
NS = 16
NW = NC * NS
SEEDS = B // NW

HD = ED // 2
EDP = 128
IDP = 256
RCH = 1024
NPAD = 100352
TS = 512
TGRID = B // TS

_MESH = dict(core_axis_name="c", subcore_axis_name="s")
_SC_LIN = pltpu.CompilerParams(use_tc_tiling_on_sc=False)
_SC_TILED = pltpu.CompilerParams(use_tc_tiling_on_sc=True)


def _build_sample_masks():
    import numpy as np
    cpu = jax.local_devices(backend="cpu")[0]
    with jax.default_device(cpu):
        skey = jax.random.key(42)
        off0, off1 = [], []
        for g in range(G):
            o0 = jax.random.randint(jax.random.fold_in(skey, g * 100 + 0),
                                    (B, S0), 0, DEG, dtype=jnp.int32)
            o1 = jax.random.randint(jax.random.fold_in(skey, g * 100 + 1),
                                    (B * S0, S1), 0, DEG, dtype=jnp.int32)
            off0.append(np.asarray(o0).T)
            off1.append(np.asarray(o1).reshape(B, S0, S1).transpose(2, 1, 0))
    import numpy as _np
    off0 = _np.stack(off0).reshape(G * S0, B // 8, 8)
    off1 = _np.stack(off1).reshape(G * S1 * S0, B // 8, 8)
    lane = _np.arange(DEG, dtype=_np.int32)
    m0 = (off0[..., None] == lane).astype(_np.float32)
    m1 = (off1[..., None] == lane).astype(_np.float32)
    return (m0.reshape(G * S0, B // 8, DEG * 8),
            m1.reshape(G * S1 * S0, B // 8, DEG * 8))


_MASK0, _MASK1 = _build_sample_masks()


def _tc_repack_user(u_t):
    def body(u_r, o_r):
        o_r[:, :ED] = lax.transpose(u_r[0], (1, 0))
        o_r[:, ED:] = jnp.zeros((RCH, EDP - ED), jnp.float32)

    return pl.pallas_call(
        body,
        grid=(G, NPAD // RCH),
        in_specs=[pl.BlockSpec((1, ED, RCH), lambda g, i: (g, 0, i))],
        out_specs=pl.BlockSpec((RCH, EDP),
                               lambda g, i: (g * (NPAD // RCH) + i, 0)),
        out_shape=jax.ShapeDtypeStruct((G * NPAD, EDP), jnp.float32),
    )(u_t)


def _tc_repack_base(b_t):
    def body(b_r, o_r):
        o_r[:, :ID] = lax.transpose(b_r[...], (1, 0))
        o_r[:, ID:] = jnp.zeros((RCH, IDP - ID), jnp.float32)

    return pl.pallas_call(
        body,
        grid=(NPAD // RCH,),
        in_specs=[pl.BlockSpec((ID, RCH), lambda i: (0, i))],
        out_specs=pl.BlockSpec((RCH, IDP), lambda i: (i, 0)),
        out_shape=jax.ShapeDtypeStruct((NPAD, IDP), jnp.float32),
    )(b_t)


def _wid():
    return lax.axis_index("s") * NC + lax.axis_index("c")


def _sc_stage_a1(nodeids, nbrs_flat):
    out_type = jax.ShapeDtypeStruct((G, B, DEG), jnp.int32)
    scratch = [
        pltpu.VMEM((SEEDS,), jnp.int32),
        pltpu.VMEM((SEEDS,), jnp.int32),
        pltpu.VMEM((SEEDS, DEG), jnp.int32),
        pltpu.SemaphoreType.DMA,
    ]

    @functools.partial(pl.kernel, out_type=out_type,
                       mesh=plsc.VectorSubcoreMesh(**_MESH),
                       compiler_params=_SC_LIN,
                       scratch_types=scratch)
    def body(nodeids_h, nbrs_h, rows0_o, seeds_v, idx_v, rows_v, sem):
        wb = _wid() * SEEDS
        pltpu.sync_copy(nodeids_h.at[pl.ds(wb, SEEDS)], seeds_v)
        for g in range(G):
            for c in range(SEEDS // 16):
                idx_v[pl.ds(c * 16, 16)] = seeds_v[pl.ds(c * 16, 16)] + g * N
            pltpu.async_copy(nbrs_h.at[idx_v], rows_v, sem).wait()
            pltpu.sync_copy(rows_v, rows0_o.at[g, pl.ds(wb, SEEDS)])

    return body(nodeids, nbrs_flat)


def _sc_stage_a2(nodeids, user128, base256):
    out_type = (
        jax.ShapeDtypeStruct((G, B, EDP), jnp.float32),
        jax.ShapeDtypeStruct((B, IDP), jnp.float32),
    )
    scratch = [
        pltpu.VMEM((SEEDS,), jnp.int32),
        pltpu.VMEM((SEEDS,), jnp.int32),
        pltpu.VMEM((SEEDS, EDP), jnp.float32),
        pltpu.VMEM((SEEDS, IDP), jnp.float32),
        pltpu.SemaphoreType.DMA,
    ]

    @functools.partial(pl.kernel, out_type=out_type,
                       mesh=plsc.VectorSubcoreMesh(**_MESH),
                       compiler_params=_SC_LIN,
                       scratch_types=scratch)
    def body(nodeids_h, user_h, base_h, f0_o, base_o,
             seeds_v, idx_v, emb_v, base_v, sem):
        wb = _wid() * SEEDS
        pltpu.sync_copy(nodeids_h.at[pl.ds(wb, SEEDS)], seeds_v)
        pltpu.async_copy(base_h.at[seeds_v], base_v, sem).wait()
        pltpu.sync_copy(base_v, base_o.at[pl.ds(wb, SEEDS)])
        for g in range(G):
            for c in range(SEEDS // 16):
                idx_v[pl.ds(c * 16, 16)] = seeds_v[pl.ds(c * 16, 16)] + g * NPAD
            pltpu.async_copy(user_h.at[idx_v], emb_v, sem).wait()
            pltpu.sync_copy(emb_v, f0_o.at[g, pl.ds(wb, SEEDS)])

    return body(nodeids, user128, base256)


def _sc_stage_b1(nbrs_flat, cur1):
    T = G * S0
    out_type = jax.ShapeDtypeStruct((T, B, DEG), jnp.int32)
    scratch = [
        pltpu.VMEM((SEEDS,), jnp.int32),
        pltpu.VMEM((SEEDS, DEG), jnp.int32),
        pltpu.SemaphoreType.DMA,
    ]

    @functools.partial(pl.kernel, out_type=out_type,
                       mesh=plsc.VectorSubcoreMesh(**_MESH),
                       compiler_params=_SC_LIN,
                       scratch_types=scratch)
    def body(nbrs_h, cur1_h, rows1_o, idx_v, rows_v, sem):
        wb = _wid() * SEEDS
        for t in range(T):
            pltpu.sync_copy(cur1_h.at[t, pl.ds(wb, SEEDS)], idx_v)
            g = t // S0
            if g:
                for c in range(SEEDS // 16):
                    idx_v[pl.ds(c * 16, 16)] = (idx_v[pl.ds(c * 16, 16)]
                                                - g * (NPAD - N))
            pltpu.async_copy(nbrs_h.at[idx_v], rows_v, sem).wait()
            pltpu.sync_copy(rows_v, rows1_o.at[t, pl.ds(wb, SEEDS)])

    return body(nbrs_flat, cur1)


def _sc_stage_b2(user128, cur1):
    T = G * S0
    out_type = jax.ShapeDtypeStruct((T, B, EDP), jnp.float32)
    scratch = [
        pltpu.VMEM((T, SEEDS), jnp.int32),
        pltpu.VMEM((SEEDS, EDP), jnp.float32),
        pltpu.VMEM((SEEDS, EDP), jnp.float32),
        pltpu.SemaphoreType.DMA,
        pltpu.SemaphoreType.DMA,
        pltpu.SemaphoreType.DMA,
    ]

    @functools.partial(pl.kernel, out_type=out_type,
                       mesh=plsc.VectorSubcoreMesh(**_MESH),
                       compiler_params=_SC_LIN,
                       scratch_types=scratch)
    def body(user_h, cur1_h, f1_o, idx_v, emb0_v, emb1_v, semg, sw0, sw1):
        wb = _wid() * SEEDS
        pltpu.sync_copy(cur1_h.at[:, pl.ds(wb, SEEDS)], idx_v)
        bufs = (emb0_v, emb1_v)
        sems = (sw0, sw1)
        wh = [None, None]
        for t in range(T):
            b = t % 2
            if wh[b] is not None:
                wh[b].wait()
            pltpu.async_copy(user_h.at[idx_v.at[t]], bufs[b], semg).wait()
            wh[b] = pltpu.async_copy(bufs[b], f1_o.at[t, pl.ds(wb, SEEDS)],
                                     sems[b])
        wh[0].wait()
        wh[1].wait()

    return body(user128, cur1)


def _sc_stage_c(user128, cur2):
    T = G * S1 * S0
    out_type = jax.ShapeDtypeStruct((T, B, EDP), jnp.float32)
    scratch = [
        pltpu.VMEM((T, SEEDS), jnp.int32),
        pltpu.VMEM((SEEDS, EDP), jnp.float32),
        pltpu.VMEM((SEEDS, EDP), jnp.float32),
        pltpu.SemaphoreType.DMA,
        pltpu.SemaphoreType.DMA,
        pltpu.SemaphoreType.DMA,
    ]

    @functools.partial(pl.kernel, out_type=out_type,
                       mesh=plsc.VectorSubcoreMesh(**_MESH),
                       compiler_params=_SC_LIN,
                       scratch_types=scratch)
    def body(user_h, cur2_h, f2_o, idx_v, emb0_v, emb1_v, semg, sw0, sw1):
        wb = _wid() * SEEDS
        pltpu.sync_copy(cur2_h.at[:, pl.ds(wb, SEEDS)], idx_v)
        bufs = (emb0_v, emb1_v)
        sems = (sw0, sw1)
        wh = [None, None]
        gh = [None, None]
        gh[0] = pltpu.async_copy(user_h.at[idx_v.at[0]], bufs[0], semg)
        for t in range(T):
            b = t % 2
            nb = (t + 1) % 2
            if t + 1 < T:
                if wh[nb] is not None:
                    wh[nb].wait()
                gh[nb] = pltpu.async_copy(user_h.at[idx_v.at[t + 1]],
                                          bufs[nb], semg)
            gh[b].wait()
            wh[b] = pltpu.async_copy(bufs[b], f2_o.at[t, pl.ds(wb, SEEDS)],
                                     sems[b])
        wh[0].wait()
        wh[1].wait()

    return body(user128, cur2)


def _sel_body(ngroups):
    TSP = TS // 8

    def body(rows_r, mask_r, out_r):
        grp = (lax.broadcasted_iota(jnp.int32, (DEG * 8, 8), 0) // DEG ==
               lax.broadcasted_iota(jnp.int32, (DEG * 8, 8), 1)
               ).astype(jnp.float32)
        for t in range(ngroups):
            if ngroups == G * S0:
                row_idx = t // S0
                bias = (t // S0) * NPAD
            else:
                g = t // (S1 * S0)
                j = t % S0
                row_idx = g * S0 + j
                bias = g * NPAD
            picked = rows_r[row_idx].astype(jnp.float32) * mask_r[t]
            val = jnp.dot(picked, grp,
                          precision=jax.lax.Precision.HIGHEST,
                          preferred_element_type=jnp.float32)
            out_r[t] = val.astype(jnp.int32) + bias
    return body


def _tc_select(rows_packed, mask, ngroups, nrows):
    TSP = TS // 8
    out = pl.pallas_call(
        _sel_body(ngroups),
        grid=(TGRID,),
        in_specs=[
            pl.BlockSpec((nrows, TSP, DEG * 8), lambda i: (0, i, 0)),
            pl.BlockSpec((ngroups, TSP, DEG * 8), lambda i: (0, i, 0)),
        ],
        out_specs=pl.BlockSpec((ngroups, TSP, 8), lambda i: (0, i, 0)),
        out_shape=jax.ShapeDtypeStruct((ngroups, B // 8, 8), jnp.int32),
    )(rows_packed, mask)
    return out.reshape(ngroups, B)


def _dense_body(f0_r, f1_r, f2_r, base_r, et_r,
                wx_r, bx_r, wn_r, bn_r, fcw_r, fcb_r,
                lng_r, lnb_r, wq_r, wk_r, wv_r, wo_r, refl_r, out_r):
    dot = functools.partial(jnp.dot, preferred_element_type=jnp.float32)
    spec = []
    for g in range(G):
        wx0a = wx_r[g * L + 0, :HD, :]
        wx0b = wx_r[g * L + 0, HD:, :]
        wn0a = wn_r[g * L + 0, :HD, :]
        wn0b = wn_r[g * L + 0, HD:, :]
        bx0 = bx_r[g * L + 0]
        bn0 = bn_r[g * L + 0]
        wx1a = wx_r[g * L + 1, :HD, :]
        wx1b = wx_r[g * L + 1, HD:, :]
        wn1a = wn_r[g * L + 1, :HD, :]
        wn1b = wn_r[g * L + 1, HD:, :]
        bx1 = bx_r[g * L + 1]
        bn1 = bn_r[g * L + 1]

        f0 = f0_r[g]
        f1 = [f1_r[g * S0 + j] for j in range(S0)]
        h1a, h1b = [], []
        for j in range(S0):
            nb2 = f2_r[(g * S1 + 0) * S0 + j]
            for j2 in range(1, S1):
                nb2 = nb2 + f2_r[(g * S1 + j2) * S0 + j]
            nb2 = nb2 * (1.0 / S1)
            xa = dot(f1[j][:, :HD], wx0a) + dot(f1[j][:, HD:ED], wx0b) + bx0
            xb = dot(nb2[:, :HD], wn0a) + dot(nb2[:, HD:ED], wn0b) + bn0
            h1a.append(jnp.maximum(xa, 0.0))
            h1b.append(jnp.maximum(xb, 0.0))
        nb1 = (f1[0] + f1[1] + f1[2]) * (1.0 / S0)
        xa = dot(f0[:, :HD], wx0a) + dot(f0[:, HD:ED], wx0b) + bx0
        xb = dot(nb1[:, :HD], wn0a) + dot(nb1[:, HD:ED], wn0b) + bn0
        h0a = jnp.maximum(xa, 0.0)
        h0b = jnp.maximum(xb, 0.0)
        nha = (h1a[0] + h1a[1] + h1a[2]) * (1.0 / S0)
        nhb = (h1b[0] + h1b[1] + h1b[2]) * (1.0 / S0)
        ya = dot(h0a, wx1a) + dot(h0b, wx1b) + bx1
        yb = dot(nha, wn1a) + dot(nhb, wn1b) + bn1
        ha = jnp.maximum(ya, 0.0)
        hb = jnp.maximum(yb, 0.0)
        nrm2 = jnp.sum(ha * ha, axis=1, keepdims=True) + \
            jnp.sum(hb * hb, axis=1, keepdims=True)
        inv = 1.0 / jnp.maximum(jnp.sqrt(nrm2), 1e-12)
        fca = fcw_r[g, :HD, :]
        fcb_w = fcw_r[g, HD:, :]
        sg = (dot(ha, fca) + dot(hb, fcb_w)) * inv + fcb_r[g]
        spec.append(sg)

    lng = lng_r[0]
    lnb = lnb_r[0]
    q, k, v = [], [], []
    for g in range(G):
        s = spec[g]
        m = jnp.mean(s, axis=1, keepdims=True)
        var = jnp.mean((s - m) * (s - m), axis=1, keepdims=True)
        qn = (s - m) / jnp.sqrt(var + 1e-6) * lng + lnb
        q.append(dot(qn, wq_r[...]))
        k.append(dot(s, wk_r[...]))
        v.append(dot(s, wv_r[...]))
    scale = 1.0 / jnp.sqrt(jnp.float32(ED))
    o = []
    for qq in range(G):
        lg = [jnp.sum(q[qq] * k[kk], axis=1, keepdims=True) * scale
              for kk in range(G)]
        mx = jnp.maximum(jnp.maximum(lg[0], lg[1]), lg[2])
        e = [jnp.exp(x - mx) for x in lg]
        ssum = e[0] + e[1] + e[2]
        att = [x / ssum for x in e]
        ov = att[0] * v[0] + att[1] * v[1] + att[2] * v[2]
        o.append(dot(ov, wo_r[...]) + spec[qq])

    et = et_r[...]
    acc = jnp.zeros((TS, OD), jnp.float32)
    for g in range(G):
        pr = dot(o[g], refl_r[g])
        acc = acc + jnp.where(et == g, pr, 0.0)
    fin = base_r[:, :ID] + acc
    nrm = jnp.sqrt(jnp.sum(fin * fin, axis=1, keepdims=True))
    out_r[...] = fin / jnp.maximum(nrm, 1e-12)


def _dense_call(f0, f1, f2, baser, et2, agg_Wx, agg_bx, agg_Wn, agg_bn,
                fc_W, fc_b, ln_g, ln_b, wq, wk, wv, w_o, reflect):
    def blk(shape, im):
        return pl.BlockSpec(shape, im)

    def full(x):
        nd = x.ndim
        return pl.BlockSpec(x.shape, lambda t, nd=nd: (0,) * nd)

    in_specs = [
        blk((G, TS, EDP), lambda t: (0, t, 0)),
        blk((G * S0, TS, EDP), lambda t: (0, t, 0)),
        blk((G * S1 * S0, TS, EDP), lambda t: (0, t, 0)),
        blk((TS, IDP), lambda t: (t, 0)),
        blk((TS, 1), lambda t: (t, 0)),
        full(agg_Wx), full(agg_bx), full(agg_Wn), full(agg_bn),
        full(fc_W), full(fc_b), full(ln_g), full(ln_b),
        full(wq), full(wk), full(wv), full(w_o), full(reflect),
    ]
    return pl.pallas_call(
        _dense_body,
        grid=(TGRID,),
        in_specs=in_specs,
        out_specs=pl.BlockSpec((TS, OD), lambda t: (t, 0)),
        out_shape=jax.ShapeDtypeStruct((B, OD), jnp.float32),
    )(f0, f1, f2, baser, et2, agg_Wx, agg_bx, agg_Wn, agg_bn,
      fc_W, fc_b, ln_g, ln_b, wq, wk, wv, w_o, reflect)


def kernel(nodeids, edgetypes, nbrs, base_embed, user_embed, agg_Wx, agg_bx,
           agg_Wn, agg_bn, fc_W, fc_b, ln_g, ln_b, wq, wk, wv, w_o, reflect):
    mask0 = jnp.asarray(_MASK0)
    mask1 = jnp.asarray(_MASK1)

    nbrs_flat = nbrs.reshape(G * N, DEG)
    user128 = _tc_repack_user(jnp.transpose(user_embed, (0, 2, 1)))
    base256 = _tc_repack_base(jnp.transpose(base_embed, (1, 0)))

    rows0 = _sc_stage_a1(nodeids, nbrs_flat)
    f0, baser = _sc_stage_a2(nodeids, user128, base256)
    cur1 = _tc_select(rows0.reshape(G, B // 8, DEG * 8),
                      mask0, G * S0, G)
    rows1 = _sc_stage_b1(nbrs_flat, cur1)
    f1 = _sc_stage_b2(user128, cur1)
    cur2 = _tc_select(rows1.reshape(G * S0, B // 8, DEG * 8),
                      mask1, G * S1 * S0, G * S0)
    f2 = _sc_stage_c(user128, cur2)

    et2 = edgetypes.reshape(B, 1)
    lng2 = ln_g.reshape(1, ED)
    lnb2 = ln_b.reshape(1, ED)
    wx = agg_Wx.reshape(G * L, ED, HD)
    bx = agg_bx.reshape(G * L, HD)
    wn = agg_Wn.reshape(G * L, ED, HD)
    bn = agg_bn.reshape(G * L, HD)

    return _dense_call(f0, f1, f2, baser, et2, wx, bx, wn, bn,
                       fc_W, fc_b, lng2, lnb2, wq, wk, wv, w_o, reflect)
